# Initial kernel scaffold; baseline (speedup 1.0000x reference)
import jax
import jax.numpy as jnp
from jax.experimental import pallas as pl


def kernel(feat_query, feat_database, params):
    raise NotImplementedError("write your pallas kernel here")


# speedup vs baseline: 3.7361x; 3.7361x over previous
import functools

import jax
import jax.numpy as jnp
from jax.experimental import pallas as pl
from jax.experimental.pallas import tpu as pltpu

FEAT = 512
K = 1024
N = K + 1
NP = 1152
DIN = 2 * K + 1
DP = 2176
HID = 2048
LR = 1e-3
EPS_N = 1e-12
EPS_IN = 1e-5


def _sim_body(feat_ref, s_ref, f_ref, n_ref):
    x = feat_ref[...]
    n2 = jnp.sum(x * x, axis=1, keepdims=True)
    n = jnp.sqrt(n2)
    F = x / jnp.maximum(n, EPS_N)
    s_ref[...] = jax.lax.dot_general(
        F, F, (((1,), (1,)), ((), ())), preferred_element_type=jnp.float32)
    f_ref[...] = F
    n_ref[...] = n


def _similarity(feat_p):
    return pl.pallas_call(
        _sim_body,
        out_shape=(
            jax.ShapeDtypeStruct((NP, NP), jnp.float32),
            jax.ShapeDtypeStruct((NP, FEAT), jnp.float32),
            jax.ShapeDtypeStruct((NP, 1), jnp.float32),
        ),
    )(feat_p)


def _dni_body(scal_ref, x_ref, w1_ref, b1_ref, w2_ref, b2_ref, w3_ref,
              b3_ref, out_ref):
    x = x_ref[...]
    mask = jax.lax.broadcasted_iota(jnp.int32, x.shape, 1) < DIN

    def inorm(h, d, w, b, msk=None):
        mean = jnp.sum(h, axis=1, keepdims=True) / d
        var = jnp.sum(h * h, axis=1, keepdims=True) / d - mean * mean
        out = (h - mean) * jax.lax.rsqrt(var + EPS_IN) * w + b
        if msk is not None:
            out = jnp.where(msk, out, 0.0)
        return out

    def mm(a, wt):
        return jax.lax.dot_general(
            a.astype(jnp.bfloat16), wt[...],
            (((1,), (0,)), ((), ())), preferred_element_type=jnp.float32)

    h = inorm(x, DIN, scal_ref[0], scal_ref[1], mask)
    h = jnp.maximum(mm(h, w1_ref) + b1_ref[...], 0.0)
    h = inorm(h, HID, scal_ref[2], scal_ref[3])
    h = jnp.maximum(mm(h, w2_ref) + b2_ref[...], 0.0)
    h = inorm(h, HID, scal_ref[4], scal_ref[5])
    out_ref[...] = mm(h, w3_ref) + b3_ref[...]


def _dni(x_p, wp):
    br = 128
    grid = (NP // br,)
    w_spec = lambda shape: pl.BlockSpec(shape, lambda i: (0, 0))
    return pl.pallas_call(
        _dni_body,
        grid=grid,
        in_specs=[
            pl.BlockSpec(memory_space=pltpu.SMEM),
            pl.BlockSpec((br, DP), lambda i: (i, 0)),
            w_spec((DP, HID)), w_spec((1, HID)),
            w_spec((HID, HID)), w_spec((1, HID)),
            w_spec((HID, DP)), w_spec((1, DP)),
        ],
        out_specs=pl.BlockSpec((br, DP), lambda i: (i, 0)),
        out_shape=jax.ShapeDtypeStruct((NP, DP), jnp.float32),
    )(wp['scal'], x_p, wp['w1t'], wp['b1'], wp['w2t'], wp['b2'],
      wp['w3t'], wp['b3'])


def _bwd_body(b_ref, f_ref, n_ref, feat_ref, out_ref):
    B = b_ref[...].astype(jnp.bfloat16)
    F = f_ref[...]
    Fb = F.astype(jnp.bfloat16)
    dF = jax.lax.dot_general(
        B, Fb, (((1,), (0,)), ((), ())), preferred_element_type=jnp.float32)
    dF += jax.lax.dot_general(
        B, Fb, (((0,), (0,)), ((), ())), preferred_element_type=jnp.float32)
    n = n_ref[...]
    dot = jnp.sum(dF * F, axis=1, keepdims=True)
    g = (dF - jnp.where(n > EPS_N, dot * F, 0.0)) / jnp.maximum(n, EPS_N)
    out_ref[...] = feat_ref[...] - LR * g


def _bwd_update(B_p, F_p, n_p, feat_p):
    return pl.pallas_call(
        _bwd_body,
        out_shape=jax.ShapeDtypeStruct((NP, FEAT), jnp.float32),
    )(B_p, F_p, n_p, feat_p)


def _stage(feat_p, wp, update_q):
    S, F, n = _similarity(feat_p)
    keys = S[:N, 1:N]
    _, p = jax.lax.top_k(keys, K)
    ip = jnp.argsort(p, axis=-1)
    sortedvals = jnp.take_along_axis(keys, p, axis=-1)
    s0 = S[0, 1:N]
    second = s0[p]
    col0 = S[0, :N]
    x = jnp.concatenate([col0[:, None], sortedvals, second], axis=1)
    x_p = jnp.pad(x, ((0, NP - N), (0, DP - DIN)))
    gl = _dni(x_p, wp)[:N, :DIN]
    glA = gl[:, 1:K + 1]
    glC = gl[:, K + 1:]
    A = jnp.take_along_axis(glA, ip, axis=-1)
    rowsum = jnp.sum(jnp.take_along_axis(glC, ip, axis=-1), axis=0)
    w = gl[:, 0] + jnp.concatenate([jnp.zeros((1,), jnp.float32),
                                    A[0] + rowsum])
    if update_q:
        B = jnp.zeros((NP, NP), jnp.float32).at[0, :N].set(w)
    else:
        B = jnp.zeros((NP, NP), jnp.float32)
        B = B.at[:N, 1:K + 1].set(A).at[0, :N].set(w)
    new_feat = _bwd_update(B, F, n, feat_p)
    if update_q:
        return jnp.concatenate([new_feat[0:1], feat_p[1:]], axis=0)
    return jnp.concatenate([feat_p[0:1], new_feat[1:]], axis=0)


def _prep_params(p):
    scal = jnp.stack([p['in1_w'], p['in1_b'], p['in2_w'], p['in2_b'],
                      p['in3_w'], p['in3_b']])
    w1t = jnp.pad(p['l1_W'].T, ((0, DP - DIN), (0, 0))).astype(jnp.bfloat16)
    w2t = p['l2_W'].T.astype(jnp.bfloat16)
    w3t = jnp.pad(p['l3_W'].T, ((0, 0), (0, DP - DIN))).astype(jnp.bfloat16)
    b1 = p['l1_b'][None, :]
    b2 = p['l2_b'][None, :]
    b3 = jnp.pad(p['l3_b'], (0, DP - DIN))[None, :]
    return dict(scal=scal, w1t=w1t, b1=b1, w2t=w2t, b2=b2, w3t=w3t, b3=b3)


def kernel(feat_query, feat_database, params):
    feat = jnp.concatenate([feat_query, feat_database], axis=0)
    feat_p = jnp.pad(feat, ((0, NP - N), (0, 0)))
    wq = _prep_params(params['query'])
    wd = _prep_params(params['database'])
    for _ in range(2):
        feat_p = _stage(feat_p, wq, True)
        feat_p = _stage(feat_p, wd, False)
    S, _, _ = _similarity(feat_p)
    return S[0, 1:N]


# speedup vs baseline: 91.8267x; 24.5785x over previous
import functools

import jax
import jax.numpy as jnp
from jax import lax
from jax.experimental import pallas as pl
from jax.experimental.pallas import tpu as pltpu
from jax.experimental.pallas import tpu_sc as plsc

FEAT = 512
K = 1024
N = K + 1
NP = 1152
DIN = 2 * K + 1
DP = 2176
HID = 2048
LR = 1e-3
EPS_N = 1e-12
EPS_IN = 1e-5

L = 16
NV = K // L
NB = 256
NW = 32
RPW = NP // NW

_MESH = plsc.VectorSubcoreMesh(core_axis_name="c", subcore_axis_name="s")
_SC_PARAMS = pltpu.CompilerParams(needs_layout_passes=False)


def _sim_body(feat_ref, s_ref, f_ref, n_ref):
    x = feat_ref[...]
    n = jnp.sqrt(jnp.sum(x * x, axis=1, keepdims=True))
    F = x / jnp.maximum(n, EPS_N)
    s_ref[...] = lax.dot_general(
        F, F, (((1,), (1,)), ((), ())), preferred_element_type=jnp.float32)
    f_ref[...] = F
    n_ref[...] = n


def _similarity(feat_p):
    return pl.pallas_call(
        _sim_body,
        out_shape=(
            jax.ShapeDtypeStruct((NP, NP), jnp.float32),
            jax.ShapeDtypeStruct((NP, FEAT), jnp.float32),
            jax.ShapeDtypeStruct((NP, 1), jnp.float32),
        ),
    )(feat_p)


def _final_body(feat_ref, out_ref):
    x = feat_ref[...]
    n = jnp.sqrt(jnp.sum(x * x, axis=1, keepdims=True))
    F = x / jnp.maximum(n, EPS_N)
    out_ref[...] = lax.dot_general(
        F[0:1], F, (((1,), (1,)), ((), ())),
        preferred_element_type=jnp.float32)


def _final_row(feat_p):
    return pl.pallas_call(
        _final_body,
        out_shape=jax.ShapeDtypeStruct((1, NP), jnp.float32),
    )(feat_p)


def _dni_body(scal_ref, x_ref, w1_ref, b1_ref, w2_ref, b2_ref, w3_ref,
              b3_ref, out_ref):
    x = x_ref[...]
    mask = lax.broadcasted_iota(jnp.int32, x.shape, 1) < DIN

    def inorm(h, d, w, b, msk=None):
        mean = jnp.sum(h, axis=1, keepdims=True) / d
        var = jnp.sum(h * h, axis=1, keepdims=True) / d - mean * mean
        out = (h - mean) * lax.rsqrt(var + EPS_IN) * w + b
        if msk is not None:
            out = jnp.where(msk, out, 0.0)
        return out

    def mm(a, w_ref):
        return lax.dot_general(
            a.astype(jnp.bfloat16), w_ref[...],
            (((1,), (1,)), ((), ())), preferred_element_type=jnp.float32)

    h = inorm(x, DIN, scal_ref[0], scal_ref[1], mask)
    h = jnp.maximum(mm(h, w1_ref) + b1_ref[...], 0.0)
    h = inorm(h, HID, scal_ref[2], scal_ref[3])
    h = jnp.maximum(mm(h, w2_ref) + b2_ref[...], 0.0)
    h = inorm(h, HID, scal_ref[4], scal_ref[5])
    out_ref[...] = mm(h, w3_ref) + b3_ref[...]


def _dni(x_p, wp):
    br = 128
    w_spec = lambda shape: pl.BlockSpec(shape, lambda i: (0, 0))
    return pl.pallas_call(
        _dni_body,
        grid=(NP // br,),
        in_specs=[
            pl.BlockSpec(memory_space=pltpu.SMEM),
            pl.BlockSpec((br, DP), lambda i: (i, 0)),
            w_spec((HID, DP)), w_spec((1, HID)),
            w_spec((HID, HID)), w_spec((1, HID)),
            w_spec((DP, HID)), w_spec((1, DP)),
        ],
        out_specs=pl.BlockSpec((br, DP), lambda i: (i, 0)),
        out_shape=jax.ShapeDtypeStruct((NP, DP), jnp.float32),
    )(wp['scal'], x_p, wp['w1'], wp['b1'], wp['w2'], wp['b2'],
      wp['w3'], wp['b3'])


def _monotone_desc(u):
    m = lax.shift_right_arithmetic(u, 31) | jnp.int32(-2 ** 31)
    return ~(u ^ m)


def _digit(k, shift):
    return lax.shift_right_logical(k, shift) & 0xFF


def _radix_argsort(ka, ia, kb, ib, hist):
    bufs = [(ka, ia), (kb, ib)]
    for pnum, shift in enumerate((0, 8, 16, 24)):
        src_k, src_i = bufs[pnum % 2]
        dst_k, dst_i = bufs[(pnum + 1) % 2]

        def zero_body(b, c):
            hist[pl.ds(b * L, L)] = jnp.zeros((L,), jnp.int32)
            return c
        lax.fori_loop(0, NB // L, zero_body, 0)

        def hist_body(j, c, src_k=src_k, shift=shift):
            d = _digit(src_k[pl.ds(j * L, L)], shift)
            occ, lastm = plsc.scan_count(d)
            plsc.addupdate_scatter(hist, [d], occ, mask=lastm)
            return c
        lax.fori_loop(0, NV, hist_body, 0)

        def scan_body(b, carry):
            v = hist[pl.ds(b * L, L)]
            cs = plsc.cumsum(v)
            hist[pl.ds(b * L, L)] = cs - v + carry
            return carry + jnp.sum(v)
        lax.fori_loop(0, NB // L, scan_body, 0)

        def perm_body(j, c, pnum=pnum, shift=shift, src_k=src_k,
                      src_i=src_i, dst_k=dst_k, dst_i=dst_i):
            k = src_k[pl.ds(j * L, L)]
            if pnum == 0:
                iv = lax.iota(jnp.int32, L) + j * L
            else:
                iv = src_i[pl.ds(j * L, L)]
            d = _digit(k, shift)
            base = plsc.load_gather(hist, [d])
            occ, lastm = plsc.scan_count(d)
            pos = base + occ - 1
            plsc.store_scatter(dst_k, [pos], k)
            plsc.store_scatter(dst_i, [pos], iv)
            plsc.addupdate_scatter(hist, [d], occ, mask=lastm)
            return c
        lax.fori_loop(0, NV, perm_body, 0)


def _make_sc_build(full_sort):
    scratch = [
        pltpu.VMEM((NP,), jnp.float32),
        pltpu.VMEM((NP,), jnp.float32),
        pltpu.VMEM((DP,), jnp.float32),
        pltpu.VMEM((K,), jnp.int32),
        pltpu.VMEM((K,), jnp.int32),
        pltpu.VMEM((K,), jnp.int32),
        pltpu.VMEM((K,), jnp.int32),
        pltpu.VMEM((NB,), jnp.int32),
    ]
    out_type = (
        jax.ShapeDtypeStruct((NP, DP), jnp.float32),
        jax.ShapeDtypeStruct((NP, K), jnp.int32),
        jax.ShapeDtypeStruct((NP, K), jnp.int32),
    )

    def body(*refs):
        if full_sort:
            (s_hbm, x_hbm, p_hbm, ip_hbm,
             srow, s0, xbuf, ka, pbuf, kb, ib, hist) = refs
            pold_hbm = None
        else:
            (s_hbm, pold_hbm, ipold_hbm, x_hbm, p_hbm, ip_hbm,
             srow, s0, xbuf, ka, pbuf, kb, ib, hist) = refs
        wid = lax.axis_index("s") * 2 + lax.axis_index("c")
        pltpu.sync_copy(s_hbm.at[0], s0)

        def zero_x(j, c):
            xbuf[pl.ds(j * L, L)] = jnp.zeros((L,), jnp.float32)
            return c
        lax.fori_loop(0, DP // L, zero_x, 0)

        def row_body(r, c):
            row = wid * RPW + r
            pltpu.sync_copy(s_hbm.at[row], srow)

            def do_sort():
                def load_body(j, c2):
                    mv = lax.iota(jnp.int32, L) + j * L
                    u = plsc.bitcast(plsc.load_gather(srow, [mv + 1]),
                                     jnp.int32)
                    ka[pl.ds(j * L, L)] = _monotone_desc(u)
                    return c2
                lax.fori_loop(0, NV, load_body, 0)
                _radix_argsort(ka, pbuf, kb, ib, hist)

            if full_sort:
                do_sort()
            else:
                @pl.when(row == 0)
                def _():
                    do_sort()

                @pl.when(row > 0)
                def _():
                    pltpu.sync_copy(pold_hbm.at[row], pbuf)

            def post_body(j, c2):
                mv = lax.iota(jnp.int32, L) + j * L
                pv = pbuf[pl.ds(j * L, L)]
                plsc.store_scatter(ib, [pv], mv)
                sv = plsc.load_gather(srow, [pv + 1])
                plsc.store_scatter(xbuf, [mv + 1], sv)
                sec = plsc.load_gather(s0, [pv + 1])
                plsc.store_scatter(xbuf, [mv + 1 + K], sec)
                return c2
            lax.fori_loop(0, NV, post_body, 0)

            lane0 = lax.iota(jnp.int32, L) == 0
            c0 = plsc.load_gather(s0, [lax.full((L,), row, jnp.int32)])
            plsc.store_scatter(xbuf, [jnp.zeros((L,), jnp.int32)], c0,
                               mask=lane0)
            pltpu.sync_copy(xbuf, x_hbm.at[row])
            pltpu.sync_copy(pbuf, p_hbm.at[row])
            pltpu.sync_copy(ib, ip_hbm.at[row])
            return c
        lax.fori_loop(0, RPW, row_body, 0)

    return functools.partial(
        pl.kernel, out_type=out_type, mesh=_MESH, scratch_types=scratch,
        compiler_params=_SC_PARAMS)(body)


_sc_build_full = _make_sc_build(True)
_sc_build_reuse = _make_sc_build(False)


def _make_sc_bwd(qmode):
    scratch = [
        pltpu.VMEM((DP,), jnp.float32),
        pltpu.VMEM((K,), jnp.int32),
        pltpu.VMEM((NP,), jnp.float32),
        pltpu.VMEM((NP,), jnp.float32),
    ]
    if qmode:
        out_type = (jax.ShapeDtypeStruct((1, NP), jnp.float32),
                    jax.ShapeDtypeStruct((NW, NP), jnp.float32))
    else:
        out_type = (jax.ShapeDtypeStruct((NP, NP), jnp.float32),
                    jax.ShapeDtypeStruct((NW, NP), jnp.float32))

    def body(gl_hbm, ip_hbm, b_hbm, part_hbm, glrow, ipbuf, browbuf, rsum):
        wid = lax.axis_index("s") * 2 + lax.axis_index("c")

        def zero_body(j, c):
            browbuf[pl.ds(j * L, L)] = jnp.zeros((L,), jnp.float32)
            rsum[pl.ds(j * L, L)] = jnp.zeros((L,), jnp.float32)
            return c
        lax.fori_loop(0, NP // L, zero_body, 0)

        def row_body(r, c):
            row = wid * RPW + r
            pltpu.sync_copy(gl_hbm.at[row], glrow)
            pltpu.sync_copy(ip_hbm.at[row], ipbuf)

            def gather_c(j, c2):
                mv = lax.iota(jnp.int32, L) + j * L
                ipv = ipbuf[pl.ds(j * L, L)]
                cv = plsc.load_gather(glrow, [ipv + 1 + K])
                old = plsc.load_gather(rsum, [mv + 1])
                plsc.store_scatter(rsum, [mv + 1], old + cv)
                return c2

            @pl.when(row < N)
            def _():
                lax.fori_loop(0, NV, gather_c, 0)

            def gather_a(j, c2):
                mv = lax.iota(jnp.int32, L) + j * L
                ipv = ipbuf[pl.ds(j * L, L)]
                av = plsc.load_gather(glrow, [ipv + 1])
                plsc.store_scatter(browbuf, [mv + 1], av)
                return c2

            if qmode:
                @pl.when(row == 0)
                def _():
                    lax.fori_loop(0, NV, gather_a, 0)
                    pltpu.sync_copy(browbuf, b_hbm.at[0])
            else:
                lax.fori_loop(0, NV, gather_a, 0)
                pltpu.sync_copy(browbuf, b_hbm.at[row])
            return c
        lax.fori_loop(0, RPW, row_body, 0)
        pltpu.sync_copy(rsum, part_hbm.at[wid])

    return functools.partial(
        pl.kernel, out_type=out_type, mesh=_MESH, scratch_types=scratch,
        compiler_params=_SC_PARAMS)(body)


_sc_bwd_q = _make_sc_bwd(True)
_sc_bwd_d = _make_sc_bwd(False)


def _bwd_d_body(b_ref, part_ref, glb_ref, f_ref, n_ref, feat_ref, out_ref):
    rowsum = jnp.sum(part_ref[...], axis=0, keepdims=True)
    glcol0 = glb_ref[...][:, 0:1]
    B = b_ref[...]
    ri = lax.broadcasted_iota(jnp.int32, (NP, NP), 0)
    ci = lax.broadcasted_iota(jnp.int32, (NP, NP), 1)
    M = B + jnp.where(ci == 0, glcol0, 0.0) + jnp.where(ri == 0, rowsum, 0.0)
    Mb = M.astype(jnp.bfloat16)
    Fb = f_ref[...].astype(jnp.bfloat16)
    dF = lax.dot_general(Mb, Fb, (((1,), (0,)), ((), ())),
                         preferred_element_type=jnp.float32)
    dF += lax.dot_general(Mb, Fb, (((0,), (0,)), ((), ())),
                          preferred_element_type=jnp.float32)
    F = f_ref[...]
    n = n_ref[...]
    dot = jnp.sum(dF * F, axis=1, keepdims=True)
    g = (dF - jnp.where(n > EPS_N, dot * F, 0.0)) / jnp.maximum(n, EPS_N)
    rif = lax.broadcasted_iota(jnp.int32, (NP, FEAT), 0)
    upd = (rif >= 1) & (rif < N)
    out_ref[...] = feat_ref[...] - LR * jnp.where(upd, g, 0.0)


def _bwd_d(Bmat, partials, glb, F, n, feat_p):
    return pl.pallas_call(
        _bwd_d_body,
        out_shape=jax.ShapeDtypeStruct((NP, FEAT), jnp.float32),
    )(Bmat, partials, glb, F, n, feat_p)


def _bwd_q_body(a_ref, part_ref, glb_ref, f_ref, n_ref, feat_ref, out_ref):
    rowsum = jnp.sum(part_ref[...], axis=0, keepdims=True)
    t = a_ref[...] + rowsum
    F = f_ref[...]
    glcol0 = glb_ref[...][:, 0:1]
    dF0 = lax.dot_general(t, F, (((1,), (0,)), ((), ())),
                          preferred_element_type=jnp.float32)
    dF0 += lax.dot_general(glcol0, F, (((0,), (0,)), ((), ())),
                           preferred_element_type=jnp.float32)
    dF0 += glb_ref[...][0:1, 0:1] * F[0:1]
    F0 = F[0:1]
    n0 = n_ref[...][0:1]
    dot = jnp.sum(dF0 * F0, axis=1, keepdims=True)
    g0 = (dF0 - jnp.where(n0 > EPS_N, dot * F0, 0.0)) / jnp.maximum(n0, EPS_N)
    out_ref[...] = feat_ref[...]
    out_ref[0:1, :] = feat_ref[0:1, :] - LR * g0


def _bwd_q(arow0, partials, glb, F, n, feat_p):
    return pl.pallas_call(
        _bwd_q_body,
        out_shape=jax.ShapeDtypeStruct((NP, FEAT), jnp.float32),
    )(arow0, partials, glb, F, n, feat_p)


def _prep_params(p):
    scal = jnp.stack([p['in1_w'], p['in1_b'], p['in2_w'], p['in2_b'],
                      p['in3_w'], p['in3_b']])
    w1 = jnp.pad(p['l1_W'], ((0, 0), (0, DP - DIN))).astype(jnp.bfloat16)
    w2 = p['l2_W'].astype(jnp.bfloat16)
    w3 = jnp.pad(p['l3_W'], ((0, DP - DIN), (0, 0))).astype(jnp.bfloat16)
    b1 = p['l1_b'][None, :]
    b2 = p['l2_b'][None, :]
    b3 = jnp.pad(p['l3_b'], (0, DP - DIN))[None, :]
    return dict(scal=scal, w1=w1, b1=b1, w2=w2, b2=b2, w3=w3, b3=b3)


def _stage(feat_p, wp, update_q, p_old, ip_old):
    S, F, n = _similarity(feat_p)
    if p_old is None:
        x_p, p, ip = _sc_build_full(S)
    else:
        x_p, p, ip = _sc_build_reuse(S, p_old, ip_old)
    gl = _dni(x_p, wp)
    glb = lax.slice(gl, (0, 0), (NP, 128))
    if update_q:
        arow0, partials = _sc_bwd_q(gl, ip)
        feat_p = _bwd_q(arow0, partials, glb, F, n, feat_p)
    else:
        Bmat, partials = _sc_bwd_d(gl, ip)
        feat_p = _bwd_d(Bmat, partials, glb, F, n, feat_p)
    return feat_p, p, ip


def kernel(feat_query, feat_database, params):
    feat = jnp.concatenate([feat_query, feat_database], axis=0)
    feat_p = jnp.pad(feat, ((0, NP - N), (0, 0)))
    wq = _prep_params(params['query'])
    wd = _prep_params(params['database'])
    feat_p, p, ip = _stage(feat_p, wq, True, None, None)
    feat_p, p, ip = _stage(feat_p, wd, False, p, ip)
    feat_p, p, ip = _stage(feat_p, wq, True, None, None)
    feat_p, p, ip = _stage(feat_p, wd, False, p, ip)
    out = _final_row(feat_p)
    return out[0, 1:N]


# speedup vs baseline: 107.8747x; 1.1748x over previous
import functools

import jax
import jax.numpy as jnp
from jax import lax
from jax.experimental import pallas as pl
from jax.experimental.pallas import tpu as pltpu
from jax.experimental.pallas import tpu_sc as plsc

FEAT = 512
K = 1024
N = K + 1
NP = 1152
DIN = 2 * K + 1
DP = 2176
HID = 2048
LR = 1e-3
EPS_N = 1e-12
EPS_IN = 1e-5

L = 16
NV = K // L
NB = 256
NW = 32
RPW = NP // NW

_MESH = plsc.VectorSubcoreMesh(core_axis_name="c", subcore_axis_name="s")
_SC_PARAMS = pltpu.CompilerParams(needs_layout_passes=False)


def _sim_body(feat_ref, s_ref, f_ref, n_ref):
    x = feat_ref[...]
    n = jnp.sqrt(jnp.sum(x * x, axis=1, keepdims=True))
    F = x / jnp.maximum(n, EPS_N)
    s_ref[...] = lax.dot_general(
        F, F, (((1,), (1,)), ((), ())), preferred_element_type=jnp.float32)
    f_ref[...] = F
    n_ref[...] = n


def _similarity(feat_p):
    return pl.pallas_call(
        _sim_body,
        out_shape=(
            jax.ShapeDtypeStruct((NP, NP), jnp.float32),
            jax.ShapeDtypeStruct((NP, FEAT), jnp.float32),
            jax.ShapeDtypeStruct((NP, 1), jnp.float32),
        ),
    )(feat_p)


def _final_body(feat_ref, out_ref):
    x = feat_ref[...]
    n = jnp.sqrt(jnp.sum(x * x, axis=1, keepdims=True))
    F = x / jnp.maximum(n, EPS_N)
    out_ref[...] = lax.dot_general(
        F[0:1], F, (((1,), (1,)), ((), ())),
        preferred_element_type=jnp.float32)


def _final_row(feat_p):
    return pl.pallas_call(
        _final_body,
        out_shape=jax.ShapeDtypeStruct((1, NP), jnp.float32),
    )(feat_p)


def _dni_body(scal_ref, x_ref, w1_ref, b1_ref, w2_ref, b2_ref, w3_ref,
              b3_ref, out_ref):
    x = x_ref[...]
    mask = lax.broadcasted_iota(jnp.int32, x.shape, 1) < DIN

    def inorm(h, d, w, b, msk=None):
        mean = jnp.sum(h, axis=1, keepdims=True) / d
        var = jnp.sum(h * h, axis=1, keepdims=True) / d - mean * mean
        out = (h - mean) * lax.rsqrt(var + EPS_IN) * w + b
        if msk is not None:
            out = jnp.where(msk, out, 0.0)
        return out

    def mm(a, w_ref):
        return lax.dot_general(
            a.astype(jnp.bfloat16), w_ref[...],
            (((1,), (1,)), ((), ())), preferred_element_type=jnp.float32)

    h = inorm(x, DIN, scal_ref[0], scal_ref[1], mask)
    h = jnp.maximum(mm(h, w1_ref) + b1_ref[...], 0.0)
    h = inorm(h, HID, scal_ref[2], scal_ref[3])
    h = jnp.maximum(mm(h, w2_ref) + b2_ref[...], 0.0)
    h = inorm(h, HID, scal_ref[4], scal_ref[5])
    out_ref[...] = mm(h, w3_ref) + b3_ref[...]


def _dni(x_p, wp):
    br = 128
    w_spec = lambda shape: pl.BlockSpec(shape, lambda i: (0, 0))
    return pl.pallas_call(
        _dni_body,
        grid=(NP // br,),
        in_specs=[
            pl.BlockSpec(memory_space=pltpu.SMEM),
            pl.BlockSpec((br, DP), lambda i: (i, 0)),
            w_spec((HID, DP)), w_spec((1, HID)),
            w_spec((HID, HID)), w_spec((1, HID)),
            w_spec((DP, HID)), w_spec((1, DP)),
        ],
        out_specs=pl.BlockSpec((br, DP), lambda i: (i, 0)),
        out_shape=jax.ShapeDtypeStruct((NP, DP), jnp.float32),
    )(wp['scal'], x_p, wp['w1'], wp['b1'], wp['w2'], wp['b2'],
      wp['w3'], wp['b3'])


def _monotone_desc(u):
    m = lax.shift_right_arithmetic(u, 31) | jnp.int32(-2 ** 31)
    return ~(u ^ m)


def _digit(k, shift):
    return lax.shift_right_logical(k, shift) & 0xFF


_SHIFTS = (0, 8, 16, 24)
_UR = 8


def _sort_row(srow, ka, ia, kb, ib, hists):
    zv = jnp.zeros((L,), jnp.int32)
    for h in hists:
        for b in range(NB // L):
            h[pl.ds(b * L, L)] = zv

    def lh_body(j, c):
        for u in range(_UR):
            off = j * (L * _UR) + u * L
            kv = plsc.load_gather(srow, [lax.iota(jnp.int32, L) + (off + 1)])
            m = _monotone_desc(plsc.bitcast(kv, jnp.int32))
            ka[pl.ds(off, L)] = m
            for pn, shift in enumerate(_SHIFTS):
                d = _digit(m, shift)
                occ, lastm = plsc.scan_count(d)
                plsc.addupdate_scatter(hists[pn], [d], occ, mask=lastm)
        return c
    lax.fori_loop(0, NV // _UR, lh_body, 0)

    bufs = [(ka, ia), (kb, ib)]
    for pn, shift in enumerate(_SHIFTS):
        hist = hists[pn]
        carry = 0
        for b in range(NB // L):
            v = hist[pl.ds(b * L, L)]
            cs = plsc.cumsum(v)
            hist[pl.ds(b * L, L)] = cs - v + carry
            carry = carry + jnp.sum(v)
        src_k, src_i = bufs[pn % 2]
        dst_k, dst_i = bufs[(pn + 1) % 2]

        def perm_body(j, c, pn=pn, shift=shift, hist=hist, src_k=src_k,
                      src_i=src_i, dst_k=dst_k, dst_i=dst_i):
            for u in range(_UR):
                off = j * (L * _UR) + u * L
                k = src_k[pl.ds(off, L)]
                if pn == 0:
                    iv = lax.iota(jnp.int32, L) + off
                else:
                    iv = src_i[pl.ds(off, L)]
                d = _digit(k, shift)
                base = plsc.load_gather(hist, [d])
                occ, lastm = plsc.scan_count(d)
                pos = base + occ - 1
                plsc.store_scatter(dst_k, [pos], k)
                plsc.store_scatter(dst_i, [pos], iv)
                plsc.addupdate_scatter(hist, [d], occ, mask=lastm)
            return c
        lax.fori_loop(0, NV // _UR, perm_body, 0)


def _make_sc_build(full_sort):
    scratch = [
        pltpu.VMEM((NP,), jnp.float32),
        pltpu.VMEM((NP,), jnp.float32),
        pltpu.VMEM((DP,), jnp.float32),
        pltpu.VMEM((K,), jnp.int32),
        pltpu.VMEM((K,), jnp.int32),
        pltpu.VMEM((K,), jnp.int32),
        pltpu.VMEM((K,), jnp.int32),
        pltpu.VMEM((NB,), jnp.int32),
        pltpu.VMEM((NB,), jnp.int32),
        pltpu.VMEM((NB,), jnp.int32),
        pltpu.VMEM((NB,), jnp.int32),
    ]
    out_type = (
        jax.ShapeDtypeStruct((NP, DP), jnp.float32),
        jax.ShapeDtypeStruct((NP, K), jnp.int32),
        jax.ShapeDtypeStruct((NP, K), jnp.int32),
    )

    def body(*refs):
        if full_sort:
            (s_hbm, x_hbm, p_hbm, ip_hbm,
             srow, s0, xbuf, ka, pbuf, kb, ib, h0, h1, h2, h3) = refs
            pold_hbm = None
        else:
            (s_hbm, pold_hbm, ipold_hbm, x_hbm, p_hbm, ip_hbm,
             srow, s0, xbuf, ka, pbuf, kb, ib, h0, h1, h2, h3) = refs
        hists = (h0, h1, h2, h3)
        wid = lax.axis_index("s") * 2 + lax.axis_index("c")
        pltpu.sync_copy(s_hbm.at[0], s0)

        def zero_x(j, c):
            xbuf[pl.ds(j * L, L)] = jnp.zeros((L,), jnp.float32)
            return c
        lax.fori_loop(0, DP // L, zero_x, 0)

        def row_body(r, c):
            row = wid * RPW + r
            pltpu.sync_copy(s_hbm.at[row], srow)

            if full_sort:
                _sort_row(srow, ka, pbuf, kb, ib, hists)
            else:
                @pl.when(row == 0)
                def _():
                    _sort_row(srow, ka, pbuf, kb, ib, hists)

                @pl.when(row > 0)
                def _():
                    pltpu.sync_copy(pold_hbm.at[row], pbuf)

            def post_body(j, c2):
                for u in range(4):
                    off = j * (L * 4) + u * L
                    mv = lax.iota(jnp.int32, L) + off
                    pv = pbuf[pl.ds(off, L)]
                    plsc.store_scatter(ib, [pv], mv)
                    sv = plsc.load_gather(srow, [pv + 1])
                    plsc.store_scatter(xbuf, [mv + 1], sv)
                    sec = plsc.load_gather(s0, [pv + 1])
                    plsc.store_scatter(xbuf, [mv + 1 + K], sec)
                return c2
            lax.fori_loop(0, NV // 4, post_body, 0)

            lane0 = lax.iota(jnp.int32, L) == 0
            c0 = plsc.load_gather(s0, [lax.full((L,), row, jnp.int32)])
            plsc.store_scatter(xbuf, [jnp.zeros((L,), jnp.int32)], c0,
                               mask=lane0)
            pltpu.sync_copy(xbuf, x_hbm.at[row])
            pltpu.sync_copy(pbuf, p_hbm.at[row])
            pltpu.sync_copy(ib, ip_hbm.at[row])
            return c
        lax.fori_loop(0, RPW, row_body, 0)

    return functools.partial(
        pl.kernel, out_type=out_type, mesh=_MESH, scratch_types=scratch,
        compiler_params=_SC_PARAMS)(body)


_sc_build_full = _make_sc_build(True)
_sc_build_reuse = _make_sc_build(False)


def _make_sc_bwd(qmode):
    scratch = [
        pltpu.VMEM((DP,), jnp.float32),
        pltpu.VMEM((K,), jnp.int32),
        pltpu.VMEM((NP,), jnp.float32),
        pltpu.VMEM((NP,), jnp.float32),
    ]
    if qmode:
        out_type = (jax.ShapeDtypeStruct((1, NP), jnp.float32),
                    jax.ShapeDtypeStruct((NW, NP), jnp.float32))
    else:
        out_type = (jax.ShapeDtypeStruct((NP, NP), jnp.float32),
                    jax.ShapeDtypeStruct((NW, NP), jnp.float32))

    def body(gl_hbm, ip_hbm, b_hbm, part_hbm, glrow, ipbuf, browbuf, rsum):
        wid = lax.axis_index("s") * 2 + lax.axis_index("c")

        def zero_body(j, c):
            browbuf[pl.ds(j * L, L)] = jnp.zeros((L,), jnp.float32)
            rsum[pl.ds(j * L, L)] = jnp.zeros((L,), jnp.float32)
            return c
        lax.fori_loop(0, NP // L, zero_body, 0)

        def row_body(r, c):
            row = wid * RPW + r
            pltpu.sync_copy(gl_hbm.at[row], glrow)
            pltpu.sync_copy(ip_hbm.at[row], ipbuf)

            def gather_c(j, c2):
                for u in range(_UR):
                    off = j * (L * _UR) + u * L
                    mv = lax.iota(jnp.int32, L) + off
                    ipv = ipbuf[pl.ds(off, L)]
                    cv = plsc.load_gather(glrow, [ipv + 1 + K])
                    old = plsc.load_gather(rsum, [mv + 1])
                    plsc.store_scatter(rsum, [mv + 1], old + cv)
                return c2

            @pl.when(row < N)
            def _():
                lax.fori_loop(0, NV // _UR, gather_c, 0)

            def gather_a(j, c2):
                for u in range(_UR):
                    off = j * (L * _UR) + u * L
                    mv = lax.iota(jnp.int32, L) + off
                    ipv = ipbuf[pl.ds(off, L)]
                    av = plsc.load_gather(glrow, [ipv + 1])
                    plsc.store_scatter(browbuf, [mv + 1], av)
                return c2

            if qmode:
                @pl.when(row == 0)
                def _():
                    lax.fori_loop(0, NV // _UR, gather_a, 0)
                    pltpu.sync_copy(browbuf, b_hbm.at[0])
            else:
                lax.fori_loop(0, NV // _UR, gather_a, 0)
                pltpu.sync_copy(browbuf, b_hbm.at[row])
            return c
        lax.fori_loop(0, RPW, row_body, 0)
        pltpu.sync_copy(rsum, part_hbm.at[wid])

    return functools.partial(
        pl.kernel, out_type=out_type, mesh=_MESH, scratch_types=scratch,
        compiler_params=_SC_PARAMS)(body)


_sc_bwd_q = _make_sc_bwd(True)
_sc_bwd_d = _make_sc_bwd(False)


def _bwd_d_body(b_ref, part_ref, glb_ref, f_ref, n_ref, feat_ref, out_ref):
    rowsum = jnp.sum(part_ref[...], axis=0, keepdims=True)
    glcol0 = glb_ref[...][:, 0:1]
    B = b_ref[...]
    ri = lax.broadcasted_iota(jnp.int32, (NP, NP), 0)
    ci = lax.broadcasted_iota(jnp.int32, (NP, NP), 1)
    M = B + jnp.where(ci == 0, glcol0, 0.0) + jnp.where(ri == 0, rowsum, 0.0)
    Mb = M.astype(jnp.bfloat16)
    Fb = f_ref[...].astype(jnp.bfloat16)
    dF = lax.dot_general(Mb, Fb, (((1,), (0,)), ((), ())),
                         preferred_element_type=jnp.float32)
    dF += lax.dot_general(Mb, Fb, (((0,), (0,)), ((), ())),
                          preferred_element_type=jnp.float32)
    F = f_ref[...]
    n = n_ref[...]
    dot = jnp.sum(dF * F, axis=1, keepdims=True)
    g = (dF - jnp.where(n > EPS_N, dot * F, 0.0)) / jnp.maximum(n, EPS_N)
    rif = lax.broadcasted_iota(jnp.int32, (NP, FEAT), 0)
    upd = (rif >= 1) & (rif < N)
    out_ref[...] = feat_ref[...] - LR * jnp.where(upd, g, 0.0)


def _bwd_d(Bmat, partials, glb, F, n, feat_p):
    return pl.pallas_call(
        _bwd_d_body,
        out_shape=jax.ShapeDtypeStruct((NP, FEAT), jnp.float32),
    )(Bmat, partials, glb, F, n, feat_p)


def _bwd_q_body(a_ref, part_ref, glb_ref, f_ref, n_ref, feat_ref, out_ref):
    rowsum = jnp.sum(part_ref[...], axis=0, keepdims=True)
    t = a_ref[...] + rowsum
    F = f_ref[...]
    glcol0 = glb_ref[...][:, 0:1]
    dF0 = lax.dot_general(t, F, (((1,), (0,)), ((), ())),
                          preferred_element_type=jnp.float32)
    dF0 += lax.dot_general(glcol0, F, (((0,), (0,)), ((), ())),
                           preferred_element_type=jnp.float32)
    dF0 += glb_ref[...][0:1, 0:1] * F[0:1]
    F0 = F[0:1]
    n0 = n_ref[...][0:1]
    dot = jnp.sum(dF0 * F0, axis=1, keepdims=True)
    g0 = (dF0 - jnp.where(n0 > EPS_N, dot * F0, 0.0)) / jnp.maximum(n0, EPS_N)
    out_ref[...] = feat_ref[...]
    out_ref[0:1, :] = feat_ref[0:1, :] - LR * g0


def _bwd_q(arow0, partials, glb, F, n, feat_p):
    return pl.pallas_call(
        _bwd_q_body,
        out_shape=jax.ShapeDtypeStruct((NP, FEAT), jnp.float32),
    )(arow0, partials, glb, F, n, feat_p)


def _prep_params(p):
    scal = jnp.stack([p['in1_w'], p['in1_b'], p['in2_w'], p['in2_b'],
                      p['in3_w'], p['in3_b']])
    w1 = jnp.pad(p['l1_W'], ((0, 0), (0, DP - DIN))).astype(jnp.bfloat16)
    w2 = p['l2_W'].astype(jnp.bfloat16)
    w3 = jnp.pad(p['l3_W'], ((0, DP - DIN), (0, 0))).astype(jnp.bfloat16)
    b1 = p['l1_b'][None, :]
    b2 = p['l2_b'][None, :]
    b3 = jnp.pad(p['l3_b'], (0, DP - DIN))[None, :]
    return dict(scal=scal, w1=w1, b1=b1, w2=w2, b2=b2, w3=w3, b3=b3)


def _stage(feat_p, wp, update_q, p_old, ip_old):
    S, F, n = _similarity(feat_p)
    if p_old is None:
        x_p, p, ip = _sc_build_full(S)
    else:
        x_p, p, ip = _sc_build_reuse(S, p_old, ip_old)
    gl = _dni(x_p, wp)
    glb = lax.slice(gl, (0, 0), (NP, 128))
    if update_q:
        arow0, partials = _sc_bwd_q(gl, ip)
        feat_p = _bwd_q(arow0, partials, glb, F, n, feat_p)
    else:
        Bmat, partials = _sc_bwd_d(gl, ip)
        feat_p = _bwd_d(Bmat, partials, glb, F, n, feat_p)
    return feat_p, p, ip


def kernel(feat_query, feat_database, params):
    feat = jnp.concatenate([feat_query, feat_database], axis=0)
    feat_p = jnp.pad(feat, ((0, NP - N), (0, 0)))
    wq = _prep_params(params['query'])
    wd = _prep_params(params['database'])
    feat_p, p, ip = _stage(feat_p, wq, True, None, None)
    feat_p, p, ip = _stage(feat_p, wd, False, p, ip)
    feat_p, p, ip = _stage(feat_p, wq, True, None, None)
    feat_p, p, ip = _stage(feat_p, wd, False, p, ip)
    out = _final_row(feat_p)
    return out[0, 1:N]


# speedup vs baseline: 136.8902x; 1.2690x over previous
import functools

import jax
import jax.numpy as jnp
from jax import lax
from jax.experimental import pallas as pl
from jax.experimental.pallas import tpu as pltpu
from jax.experimental.pallas import tpu_sc as plsc

FEAT = 512
K = 1024
N = K + 1
NP = 1152
DIN = 2 * K + 1
DP = 2176
HID = 2048
LR = 1e-3
EPS_N = 1e-12
EPS_IN = 1e-5

L = 16
NV = K // L
NB = 256
NW = 32
RPW = NP // NW

_MESH = plsc.VectorSubcoreMesh(core_axis_name="c", subcore_axis_name="s")
_SC_PARAMS = pltpu.CompilerParams(needs_layout_passes=False)


def _sim_body(feat_ref, s_ref, f_ref, n_ref):
    x = feat_ref[...]
    n = jnp.sqrt(jnp.sum(x * x, axis=1, keepdims=True))
    F = x / jnp.maximum(n, EPS_N)
    s_ref[...] = lax.dot_general(
        F, F, (((1,), (1,)), ((), ())), preferred_element_type=jnp.float32)
    f_ref[...] = F
    n_ref[...] = n


def _similarity(feat_p):
    return pl.pallas_call(
        _sim_body,
        out_shape=(
            jax.ShapeDtypeStruct((NP, NP), jnp.float32),
            jax.ShapeDtypeStruct((NP, FEAT), jnp.float32),
            jax.ShapeDtypeStruct((NP, 1), jnp.float32),
        ),
    )(feat_p)


def _final_body(feat_ref, out_ref):
    x = feat_ref[...]
    n = jnp.sqrt(jnp.sum(x * x, axis=1, keepdims=True))
    F = x / jnp.maximum(n, EPS_N)
    out_ref[...] = lax.dot_general(
        F[0:1], F, (((1,), (1,)), ((), ())),
        preferred_element_type=jnp.float32)


def _final_row(feat_p):
    return pl.pallas_call(
        _final_body,
        out_shape=jax.ShapeDtypeStruct((1, NP), jnp.float32),
    )(feat_p)


def _dni_body(scal_ref, x_ref, w1_ref, b1_ref, w2_ref, b2_ref, w3_ref,
              b3_ref, out_ref):
    x = x_ref[...]
    mask = lax.broadcasted_iota(jnp.int32, x.shape, 1) < DIN

    def inorm(h, d, w, b, msk=None):
        mean = jnp.sum(h, axis=1, keepdims=True) / d
        var = jnp.sum(h * h, axis=1, keepdims=True) / d - mean * mean
        out = (h - mean) * lax.rsqrt(var + EPS_IN) * w + b
        if msk is not None:
            out = jnp.where(msk, out, 0.0)
        return out

    def mm(a, w_ref):
        return lax.dot_general(
            a.astype(jnp.bfloat16), w_ref[...],
            (((1,), (1,)), ((), ())), preferred_element_type=jnp.float32)

    h = inorm(x, DIN, scal_ref[0], scal_ref[1], mask)
    h = jnp.maximum(mm(h, w1_ref) + b1_ref[...], 0.0)
    h = inorm(h, HID, scal_ref[2], scal_ref[3])
    h = jnp.maximum(mm(h, w2_ref) + b2_ref[...], 0.0)
    h = inorm(h, HID, scal_ref[4], scal_ref[5])
    out_ref[...] = mm(h, w3_ref) + b3_ref[...]


def _dni(x_p, wp):
    br = 128
    w_spec = lambda shape: pl.BlockSpec(shape, lambda i: (0, 0))
    return pl.pallas_call(
        _dni_body,
        grid=(NP // br,),
        in_specs=[
            pl.BlockSpec(memory_space=pltpu.SMEM),
            pl.BlockSpec((br, DP), lambda i: (i, 0)),
            w_spec((HID, DP)), w_spec((1, HID)),
            w_spec((HID, HID)), w_spec((1, HID)),
            w_spec((DP, HID)), w_spec((1, DP)),
        ],
        out_specs=pl.BlockSpec((br, DP), lambda i: (i, 0)),
        out_shape=jax.ShapeDtypeStruct((NP, DP), jnp.float32),
    )(wp['scal'], x_p, wp['w1'], wp['b1'], wp['w2'], wp['b2'],
      wp['w3'], wp['b3'])


def _monotone_desc(u):
    m = lax.shift_right_arithmetic(u, 31) | jnp.int32(-2 ** 31)
    return ~(u ^ m)


def _digit(k, shift):
    return lax.shift_right_logical(k, shift) & 0xFF


_SHIFTS = (0, 8, 16, 24)
_UR = 8


def _sort_row(srow, ka, ia, kb, ib, hists):
    zv = jnp.zeros((L,), jnp.int32)
    for h in hists:
        for b in range(NB // L):
            h[pl.ds(b * L, L)] = zv

    def lh_body(j, c):
        for u in range(_UR):
            off = j * (L * _UR) + u * L
            kv = plsc.load_gather(srow, [lax.iota(jnp.int32, L) + (off + 1)])
            m = _monotone_desc(plsc.bitcast(kv, jnp.int32))
            ka[pl.ds(off, L)] = m
            for pn, shift in enumerate(_SHIFTS):
                d = _digit(m, shift)
                occ, lastm = plsc.scan_count(d)
                plsc.addupdate_scatter(hists[pn], [d], occ, mask=lastm)
        return c
    lax.fori_loop(0, NV // _UR, lh_body, 0)

    bufs = [(ka, ia), (kb, ib)]
    for pn, shift in enumerate(_SHIFTS):
        hist = hists[pn]
        carry = 0
        for b in range(NB // L):
            v = hist[pl.ds(b * L, L)]
            cs = plsc.cumsum(v)
            hist[pl.ds(b * L, L)] = cs - v + carry
            carry = carry + jnp.sum(v)
        src_k, src_i = bufs[pn % 2]
        dst_k, dst_i = bufs[(pn + 1) % 2]

        def perm_body(j, c, pn=pn, shift=shift, hist=hist, src_k=src_k,
                      src_i=src_i, dst_k=dst_k, dst_i=dst_i):
            for u in range(_UR):
                off = j * (L * _UR) + u * L
                k = src_k[pl.ds(off, L)]
                if pn == 0:
                    iv = lax.iota(jnp.int32, L) + off
                else:
                    iv = src_i[pl.ds(off, L)]
                d = _digit(k, shift)
                base = plsc.load_gather(hist, [d])
                occ, lastm = plsc.scan_count(d)
                pos = base + occ - 1
                plsc.store_scatter(dst_k, [pos], k)
                plsc.store_scatter(dst_i, [pos], iv)
                plsc.addupdate_scatter(hist, [d], occ, mask=lastm)
            return c
        lax.fori_loop(0, NV // _UR, perm_body, 0)


def _make_sc_build(full_sort):
    scratch = [
        pltpu.VMEM((NP,), jnp.float32),
        pltpu.VMEM((NP,), jnp.float32),
        pltpu.VMEM((NP,), jnp.float32),
        pltpu.VMEM((DP,), jnp.float32),
        pltpu.VMEM((DP,), jnp.float32),
        pltpu.VMEM((K,), jnp.int32),
        pltpu.VMEM((K,), jnp.int32),
        pltpu.VMEM((K,), jnp.int32),
        pltpu.VMEM((K,), jnp.int32),
        pltpu.VMEM((K,), jnp.int32),
        pltpu.VMEM((K,), jnp.int32),
        pltpu.VMEM((K,), jnp.int32),
        pltpu.VMEM((NB,), jnp.int32),
        pltpu.VMEM((NB,), jnp.int32),
        pltpu.VMEM((NB,), jnp.int32),
        pltpu.VMEM((NB,), jnp.int32),
        pltpu.SemaphoreType.DMA,
        pltpu.SemaphoreType.DMA,
        pltpu.SemaphoreType.DMA,
        pltpu.SemaphoreType.DMA,
    ]
    if full_sort:
        out_type = (
            jax.ShapeDtypeStruct((NP, DP), jnp.float32),
            jax.ShapeDtypeStruct((NP, K), jnp.int32),
            jax.ShapeDtypeStruct((NP, K), jnp.int32),
        )
    else:
        out_type = (
            jax.ShapeDtypeStruct((NP, DP), jnp.float32),
            jax.ShapeDtypeStruct((1, K), jnp.int32),
            jax.ShapeDtypeStruct((1, K), jnp.int32),
        )

    def body(*refs):
        if full_sort:
            (s_hbm, x_hbm, p_hbm, ip_hbm, *sc) = refs
            pold_hbm = None
        else:
            (s_hbm, pold_hbm, x_hbm, p_hbm, ip_hbm, *sc) = refs
        (srA, srB, s0, xA, xB, pA, pB, ipA, ipB, ka, kb, ibs,
         h0, h1, h2, h3, inA, inB, outA, outB) = sc
        hists = (h0, h1, h2, h3)
        srow = [srA, srB]
        xb = [xA, xB]
        pb = [pA, pB]
        ipb = [ipA, ipB]
        insem = [inA, inB]
        outsem = [outA, outB]
        wid = lax.axis_index("s") * 2 + lax.axis_index("c")
        base = wid * RPW
        pltpu.sync_copy(s_hbm.at[0], s0)

        def zero_x(j, c):
            zv = jnp.zeros((L,), jnp.float32)
            xA[pl.ds(j * L, L)] = zv
            xB[pl.ds(j * L, L)] = zv
            return c
        lax.fori_loop(0, DP // L, zero_x, 0)

        def issue_in(b, row):
            pltpu.async_copy(s_hbm.at[row], srow[b], insem[b])
            if not full_sort:
                @pl.when(row > 0)
                def _():
                    pltpu.async_copy(pold_hbm.at[row], pb[b], insem[b])

        def wait_in(b, row):
            pltpu.make_async_copy(s_hbm.at[row], srow[b], insem[b]).wait()
            if not full_sort:
                @pl.when(row > 0)
                def _():
                    pltpu.make_async_copy(
                        pold_hbm.at[row], pb[b], insem[b]).wait()

        def issue_out(b, row):
            pltpu.async_copy(xb[b], x_hbm.at[row], outsem[b])
            if full_sort:
                pltpu.async_copy(pb[b], p_hbm.at[row], outsem[b])
                pltpu.async_copy(ipb[b], ip_hbm.at[row], outsem[b])

        def wait_out(b, row):
            pltpu.make_async_copy(xb[b], x_hbm.at[row], outsem[b]).wait()
            if full_sort:
                pltpu.make_async_copy(pb[b], p_hbm.at[row], outsem[b]).wait()
                pltpu.make_async_copy(
                    ipb[b], ip_hbm.at[row], outsem[b]).wait()

        issue_in(0, base)
        issue_in(1, base + 1)

        def g_body(g, c):
            for b in range(2):
                row = base + 2 * g + b
                wait_in(b, row)

                @pl.when(g > 0)
                def _(b=b, row=row):
                    wait_out(b, row)

                if full_sort:
                    _sort_row(srow[b], ka, pb[b], kb, ibs, hists)
                else:
                    @pl.when(row == 0)
                    def _(b=b):
                        _sort_row(srow[b], ka, pb[b], kb, ibs, hists)

                def post_body(j, c2, b=b):
                    for u in range(4):
                        off = j * (L * 4) + u * L
                        mv = lax.iota(jnp.int32, L) + off
                        pv = pb[b][pl.ds(off, L)]
                        if full_sort:
                            plsc.store_scatter(ipb[b], [pv], mv)
                        sv = plsc.load_gather(srow[b], [pv + 1])
                        plsc.store_scatter(xb[b], [mv + 1], sv)
                        sec = plsc.load_gather(s0, [pv + 1])
                        plsc.store_scatter(xb[b], [mv + 1 + K], sec)
                    return c2
                lax.fori_loop(0, NV // 4, post_body, 0)

                lane0 = lax.iota(jnp.int32, L) == 0
                c0 = plsc.load_gather(s0, [lax.full((L,), row, jnp.int32)])
                plsc.store_scatter(xb[b], [jnp.zeros((L,), jnp.int32)], c0,
                                   mask=lane0)

                if not full_sort:
                    @pl.when(row == 0)
                    def _(b=b):
                        def ip_body(j, c3):
                            for u in range(4):
                                off = j * (L * 4) + u * L
                                mv = lax.iota(jnp.int32, L) + off
                                pv = pb[b][pl.ds(off, L)]
                                plsc.store_scatter(ipb[b], [pv], mv)
                            return c3
                        lax.fori_loop(0, NV // 4, ip_body, 0)
                        pltpu.sync_copy(pb[b], p_hbm.at[0])
                        pltpu.sync_copy(ipb[b], ip_hbm.at[0])

                issue_out(b, row)

                @pl.when(g < RPW // 2 - 1)
                def _(b=b, row=row):
                    issue_in(b, row + 2)
            return c
        lax.fori_loop(0, RPW // 2, g_body, 0)
        for b in range(2):
            wait_out(b, base + RPW - 2 + b)

    return functools.partial(
        pl.kernel, out_type=out_type, mesh=_MESH, scratch_types=scratch,
        compiler_params=_SC_PARAMS)(body)


_sc_build_full = _make_sc_build(True)
_sc_build_reuse = _make_sc_build(False)


def _make_sc_bwd(qmode, split_ip):
    scratch = [
        pltpu.VMEM((DP,), jnp.float32),
        pltpu.VMEM((DP,), jnp.float32),
        pltpu.VMEM((K,), jnp.int32),
        pltpu.VMEM((K,), jnp.int32),
        pltpu.VMEM((NP,), jnp.float32),
        pltpu.VMEM((NP,), jnp.float32),
        pltpu.VMEM((NP,), jnp.float32),
        pltpu.SemaphoreType.DMA,
        pltpu.SemaphoreType.DMA,
        pltpu.SemaphoreType.DMA,
        pltpu.SemaphoreType.DMA,
    ]
    if qmode:
        out_type = (jax.ShapeDtypeStruct((1, NP), jnp.float32),
                    jax.ShapeDtypeStruct((NW, NP), jnp.float32))
    else:
        out_type = (jax.ShapeDtypeStruct((NP, NP), jnp.float32),
                    jax.ShapeDtypeStruct((NW, NP), jnp.float32))

    def body(*refs):
        if split_ip:
            (gl_hbm, ip_hbm, ip0_hbm, b_hbm, part_hbm, *sc) = refs
        else:
            (gl_hbm, ip_hbm, b_hbm, part_hbm, *sc) = refs
            ip0_hbm = None
        (glA, glB, ipbA, ipbB, brA, brB, rsum,
         inA, inB, outA, outB) = sc
        glrow = [glA, glB]
        ipbuf = [ipbA, ipbB]
        brow = [brA, brB]
        insem = [inA, inB]
        outsem = [outA, outB]
        wid = lax.axis_index("s") * 2 + lax.axis_index("c")
        base = wid * RPW

        def zero_body(j, c):
            zv = jnp.zeros((L,), jnp.float32)
            brA[pl.ds(j * L, L)] = zv
            brB[pl.ds(j * L, L)] = zv
            rsum[pl.ds(j * L, L)] = zv
            return c
        lax.fori_loop(0, NP // L, zero_body, 0)

        def issue_in(b, row):
            pltpu.async_copy(gl_hbm.at[row], glrow[b], insem[b])
            if split_ip:
                @pl.when(row == 0)
                def _():
                    pltpu.async_copy(ip0_hbm.at[0], ipbuf[b], insem[b])

                @pl.when(row > 0)
                def _():
                    pltpu.async_copy(ip_hbm.at[row], ipbuf[b], insem[b])
            else:
                pltpu.async_copy(ip_hbm.at[row], ipbuf[b], insem[b])

        def wait_in(b, row):
            pltpu.make_async_copy(gl_hbm.at[row], glrow[b], insem[b]).wait()
            pltpu.make_async_copy(ip_hbm.at[row], ipbuf[b], insem[b]).wait()

        issue_in(0, base)
        issue_in(1, base + 1)

        def g_body(g, c):
            for b in range(2):
                row = base + 2 * g + b
                wait_in(b, row)
                if not qmode:
                    @pl.when(g > 0)
                    def _(b=b, row=row):
                        pltpu.make_async_copy(
                            brow[b], b_hbm.at[row], outsem[b]).wait()

                def gather_c(j, c2, b=b):
                    for u in range(_UR):
                        off = j * (L * _UR) + u * L
                        mv = lax.iota(jnp.int32, L) + off
                        ipv = ipbuf[b][pl.ds(off, L)]
                        cv = plsc.load_gather(glrow[b], [ipv + 1 + K])
                        old = plsc.load_gather(rsum, [mv + 1])
                        plsc.store_scatter(rsum, [mv + 1], old + cv)
                    return c2

                @pl.when(row < N)
                def _(b=b, row=row):
                    lax.fori_loop(0, NV // _UR, gather_c, 0)

                def gather_a(j, c2, b=b):
                    for u in range(_UR):
                        off = j * (L * _UR) + u * L
                        mv = lax.iota(jnp.int32, L) + off
                        ipv = ipbuf[b][pl.ds(off, L)]
                        av = plsc.load_gather(glrow[b], [ipv + 1])
                        plsc.store_scatter(brow[b], [mv + 1], av)
                    return c2

                if qmode:
                    @pl.when(row == 0)
                    def _(b=b):
                        lax.fori_loop(0, NV // _UR, gather_a, 0)
                        pltpu.sync_copy(brow[b], b_hbm.at[0])
                else:
                    lax.fori_loop(0, NV // _UR, gather_a, 0)
                    pltpu.async_copy(brow[b], b_hbm.at[row], outsem[b])

                @pl.when(g < RPW // 2 - 1)
                def _(b=b, row=row):
                    issue_in(b, row + 2)
            return c
        lax.fori_loop(0, RPW // 2, g_body, 0)
        if not qmode:
            for b in range(2):
                pltpu.make_async_copy(
                    brow[b], b_hbm.at[base + RPW - 2 + b], outsem[b]).wait()
        pltpu.sync_copy(rsum, part_hbm.at[wid])

    return functools.partial(
        pl.kernel, out_type=out_type, mesh=_MESH, scratch_types=scratch,
        compiler_params=_SC_PARAMS)(body)


_sc_bwd_q = _make_sc_bwd(True, False)
_sc_bwd_d = _make_sc_bwd(False, True)


def _bwd_d_body(b_ref, part_ref, glb_ref, f_ref, n_ref, feat_ref, out_ref):
    rowsum = jnp.sum(part_ref[...], axis=0, keepdims=True)
    glcol0 = glb_ref[...][:, 0:1]
    B = b_ref[...]
    ri = lax.broadcasted_iota(jnp.int32, (NP, NP), 0)
    ci = lax.broadcasted_iota(jnp.int32, (NP, NP), 1)
    M = B + jnp.where(ci == 0, glcol0, 0.0) + jnp.where(ri == 0, rowsum, 0.0)
    Mb = M.astype(jnp.bfloat16)
    Fb = f_ref[...].astype(jnp.bfloat16)
    dF = lax.dot_general(Mb, Fb, (((1,), (0,)), ((), ())),
                         preferred_element_type=jnp.float32)
    dF += lax.dot_general(Mb, Fb, (((0,), (0,)), ((), ())),
                          preferred_element_type=jnp.float32)
    F = f_ref[...]
    n = n_ref[...]
    dot = jnp.sum(dF * F, axis=1, keepdims=True)
    g = (dF - jnp.where(n > EPS_N, dot * F, 0.0)) / jnp.maximum(n, EPS_N)
    rif = lax.broadcasted_iota(jnp.int32, (NP, FEAT), 0)
    upd = (rif >= 1) & (rif < N)
    out_ref[...] = feat_ref[...] - LR * jnp.where(upd, g, 0.0)


def _bwd_d(Bmat, partials, glb, F, n, feat_p):
    return pl.pallas_call(
        _bwd_d_body,
        out_shape=jax.ShapeDtypeStruct((NP, FEAT), jnp.float32),
    )(Bmat, partials, glb, F, n, feat_p)


def _bwd_q_body(a_ref, part_ref, glb_ref, f_ref, n_ref, feat_ref, out_ref):
    rowsum = jnp.sum(part_ref[...], axis=0, keepdims=True)
    t = a_ref[...] + rowsum
    F = f_ref[...]
    glcol0 = glb_ref[...][:, 0:1]
    dF0 = lax.dot_general(t, F, (((1,), (0,)), ((), ())),
                          preferred_element_type=jnp.float32)
    dF0 += lax.dot_general(glcol0, F, (((0,), (0,)), ((), ())),
                           preferred_element_type=jnp.float32)
    dF0 += glb_ref[...][0:1, 0:1] * F[0:1]
    F0 = F[0:1]
    n0 = n_ref[...][0:1]
    dot = jnp.sum(dF0 * F0, axis=1, keepdims=True)
    g0 = (dF0 - jnp.where(n0 > EPS_N, dot * F0, 0.0)) / jnp.maximum(n0, EPS_N)
    out_ref[...] = feat_ref[...]
    out_ref[0:1, :] = feat_ref[0:1, :] - LR * g0


def _bwd_q(arow0, partials, glb, F, n, feat_p):
    return pl.pallas_call(
        _bwd_q_body,
        out_shape=jax.ShapeDtypeStruct((NP, FEAT), jnp.float32),
    )(arow0, partials, glb, F, n, feat_p)


def _prep_params(p):
    scal = jnp.stack([p['in1_w'], p['in1_b'], p['in2_w'], p['in2_b'],
                      p['in3_w'], p['in3_b']])
    w1 = jnp.pad(p['l1_W'], ((0, 0), (0, DP - DIN))).astype(jnp.bfloat16)
    w2 = p['l2_W'].astype(jnp.bfloat16)
    w3 = jnp.pad(p['l3_W'], ((0, DP - DIN), (0, 0))).astype(jnp.bfloat16)
    b1 = p['l1_b'][None, :]
    b2 = p['l2_b'][None, :]
    b3 = jnp.pad(p['l3_b'], (0, DP - DIN))[None, :]
    return dict(scal=scal, w1=w1, b1=b1, w2=w2, b2=b2, w3=w3, b3=b3)


def _stage_q(feat_p, wp):
    S, F, n = _similarity(feat_p)
    x_p, p, ip = _sc_build_full(S)
    gl = _dni(x_p, wp)
    glb = lax.slice(gl, (0, 0), (NP, 128))
    arow0, partials = _sc_bwd_q(gl, ip)
    feat_p = _bwd_q(arow0, partials, glb, F, n, feat_p)
    return feat_p, p, ip


def _stage_d(feat_p, wp, p_old, ip_old):
    S, F, n = _similarity(feat_p)
    x_p, p0, ip0 = _sc_build_reuse(S, p_old)
    gl = _dni(x_p, wp)
    glb = lax.slice(gl, (0, 0), (NP, 128))
    Bmat, partials = _sc_bwd_d(gl, ip_old, ip0)
    feat_p = _bwd_d(Bmat, partials, glb, F, n, feat_p)
    return feat_p


def kernel(feat_query, feat_database, params):
    feat = jnp.concatenate([feat_query, feat_database], axis=0)
    feat_p = jnp.pad(feat, ((0, NP - N), (0, 0)))
    wq = _prep_params(params['query'])
    wd = _prep_params(params['database'])
    feat_p, p, ip = _stage_q(feat_p, wq)
    feat_p = _stage_d(feat_p, wd, p, ip)
    feat_p, p, ip = _stage_q(feat_p, wq)
    feat_p = _stage_d(feat_p, wd, p, ip)
    out = _final_row(feat_p)
    return out[0, 1:N]


# speedup vs baseline: 143.1113x; 1.0454x over previous
import functools

import jax
import jax.numpy as jnp
from jax import lax
from jax.experimental import pallas as pl
from jax.experimental.pallas import tpu as pltpu
from jax.experimental.pallas import tpu_sc as plsc

FEAT = 512
K = 1024
N = K + 1
NP = 1152
DIN = 2 * K + 1
DP = 2176
HID = 2048
LR = 1e-3
EPS_N = 1e-12
EPS_IN = 1e-5

L = 16
NV = K // L
NB = 256
NW = 32
RPW = NP // NW

_MESH = plsc.VectorSubcoreMesh(core_axis_name="c", subcore_axis_name="s")
_SC_PARAMS = pltpu.CompilerParams(needs_layout_passes=False)


def _sim_body(feat_ref, s_ref, f_ref, n_ref):
    x = feat_ref[...]
    n = jnp.sqrt(jnp.sum(x * x, axis=1, keepdims=True))
    F = x / jnp.maximum(n, EPS_N)
    s_ref[...] = lax.dot_general(
        F, F, (((1,), (1,)), ((), ())), preferred_element_type=jnp.float32)
    f_ref[...] = F
    n_ref[...] = n


def _similarity(feat_p):
    return pl.pallas_call(
        _sim_body,
        out_shape=(
            jax.ShapeDtypeStruct((NP, NP), jnp.float32),
            jax.ShapeDtypeStruct((NP, FEAT), jnp.float32),
            jax.ShapeDtypeStruct((NP, 1), jnp.float32),
        ),
    )(feat_p)


def _final_body(feat_ref, out_ref):
    x = feat_ref[...]
    n = jnp.sqrt(jnp.sum(x * x, axis=1, keepdims=True))
    F = x / jnp.maximum(n, EPS_N)
    out_ref[...] = lax.dot_general(
        F[0:1], F, (((1,), (1,)), ((), ())),
        preferred_element_type=jnp.float32)


def _final_row(feat_p):
    return pl.pallas_call(
        _final_body,
        out_shape=jax.ShapeDtypeStruct((1, NP), jnp.float32),
    )(feat_p)


def _dni_body(scal_ref, x_ref, w1_ref, b1_ref, w2_ref, b2_ref, w3_ref,
              b3_ref, out_ref):
    x = x_ref[...]
    mask = lax.broadcasted_iota(jnp.int32, x.shape, 1) < DIN

    def inorm(h, d, w, b, msk=None):
        mean = jnp.sum(h, axis=1, keepdims=True) / d
        var = jnp.sum(h * h, axis=1, keepdims=True) / d - mean * mean
        out = (h - mean) * lax.rsqrt(var + EPS_IN) * w + b
        if msk is not None:
            out = jnp.where(msk, out, 0.0)
        return out

    def mm(a, w_ref):
        return lax.dot_general(
            a.astype(jnp.bfloat16), w_ref[...],
            (((1,), (1,)), ((), ())), preferred_element_type=jnp.float32)

    h = inorm(x, DIN, scal_ref[0], scal_ref[1], mask)
    h = jnp.maximum(mm(h, w1_ref) + b1_ref[...], 0.0)
    h = inorm(h, HID, scal_ref[2], scal_ref[3])
    h = jnp.maximum(mm(h, w2_ref) + b2_ref[...], 0.0)
    h = inorm(h, HID, scal_ref[4], scal_ref[5])
    out_ref[...] = mm(h, w3_ref) + b3_ref[...]


def _dni(x_p, wp):
    br = 128
    w_spec = lambda shape: pl.BlockSpec(shape, lambda i: (0, 0))
    return pl.pallas_call(
        _dni_body,
        grid=(NP // br,),
        in_specs=[
            pl.BlockSpec(memory_space=pltpu.SMEM),
            pl.BlockSpec((br, DP), lambda i: (i, 0)),
            w_spec((HID, DP)), w_spec((1, HID)),
            w_spec((HID, HID)), w_spec((1, HID)),
            w_spec((DP, HID)), w_spec((1, DP)),
        ],
        out_specs=pl.BlockSpec((br, DP), lambda i: (i, 0)),
        out_shape=jax.ShapeDtypeStruct((NP, DP), jnp.float32),
    )(wp['scal'], x_p, wp['w1'], wp['b1'], wp['w2'], wp['b2'],
      wp['w3'], wp['b3'])


def _monotone_desc(u):
    m = lax.shift_right_arithmetic(u, 31) | jnp.int32(-2 ** 31)
    return ~(u ^ m)


def _digit(k, shift):
    return lax.shift_right_logical(k, shift) & 0xFF


_SHIFTS = (0, 8, 16, 24)
_UR = 8


def _sort_row(srow, ka, ia, kb, ib, hists):
    zv = jnp.zeros((L,), jnp.int32)
    for h in hists:
        for b in range(NB // L):
            h[pl.ds(b * L, L)] = zv

    def lh_body(j, c):
        for u in range(_UR):
            off = j * (L * _UR) + u * L
            kv = plsc.load_gather(srow, [lax.iota(jnp.int32, L) + (off + 1)])
            m = _monotone_desc(plsc.bitcast(kv, jnp.int32))
            ka[pl.ds(off, L)] = m
            for pn, shift in enumerate(_SHIFTS):
                d = _digit(m, shift)
                occ, lastm = plsc.scan_count(d)
                plsc.addupdate_scatter(hists[pn], [d], occ, mask=lastm)
        return c
    lax.fori_loop(0, NV // _UR, lh_body, 0)

    bufs = [(ka, ia), (kb, ib)]
    for pn, shift in enumerate(_SHIFTS):
        hist = hists[pn]
        carry = 0
        for b in range(NB // L):
            v = hist[pl.ds(b * L, L)]
            cs = plsc.cumsum(v)
            hist[pl.ds(b * L, L)] = cs - v + carry
            carry = carry + jnp.sum(v)
        src_k, src_i = bufs[pn % 2]
        dst_k, dst_i = bufs[(pn + 1) % 2]

        def perm_body(j, c, pn=pn, shift=shift, hist=hist, src_k=src_k,
                      src_i=src_i, dst_k=dst_k, dst_i=dst_i):
            for u in range(_UR):
                off = j * (L * _UR) + u * L
                k = src_k[pl.ds(off, L)]
                if pn == 0:
                    iv = lax.iota(jnp.int32, L) + off
                else:
                    iv = src_i[pl.ds(off, L)]
                d = _digit(k, shift)
                base = plsc.load_gather(hist, [d])
                occ, lastm = plsc.scan_count(d)
                pos = base + occ - 1
                plsc.store_scatter(dst_k, [pos], k)
                plsc.store_scatter(dst_i, [pos], iv)
                plsc.addupdate_scatter(hist, [d], occ, mask=lastm)
            return c
        lax.fori_loop(0, NV // _UR, perm_body, 0)


def _make_sc_build(full_sort):
    scratch = [
        pltpu.VMEM((NP,), jnp.float32),
        pltpu.VMEM((NP,), jnp.float32),
        pltpu.VMEM((NP,), jnp.float32),
        pltpu.VMEM((DP,), jnp.float32),
        pltpu.VMEM((DP,), jnp.float32),
        pltpu.VMEM((K,), jnp.int32),
        pltpu.VMEM((K,), jnp.int32),
        pltpu.VMEM((K,), jnp.int32),
        pltpu.VMEM((K,), jnp.int32),
        pltpu.VMEM((K,), jnp.int32),
        pltpu.VMEM((K,), jnp.int32),
        pltpu.VMEM((K,), jnp.int32),
        pltpu.VMEM((NB,), jnp.int32),
        pltpu.VMEM((NB,), jnp.int32),
        pltpu.VMEM((NB,), jnp.int32),
        pltpu.VMEM((NB,), jnp.int32),
        pltpu.SemaphoreType.DMA,
        pltpu.SemaphoreType.DMA,
        pltpu.SemaphoreType.DMA,
        pltpu.SemaphoreType.DMA,
    ]
    if full_sort:
        out_type = (
            jax.ShapeDtypeStruct((NP, DP), jnp.float32),
            jax.ShapeDtypeStruct((NP, K), jnp.int32),
            jax.ShapeDtypeStruct((NP, K), jnp.int32),
        )
    else:
        out_type = (
            jax.ShapeDtypeStruct((NP, DP), jnp.float32),
            jax.ShapeDtypeStruct((1, K), jnp.int32),
            jax.ShapeDtypeStruct((1, K), jnp.int32),
        )

    def body(*refs):
        if full_sort:
            (s_hbm, x_hbm, p_hbm, ip_hbm, *sc) = refs
            pold_hbm = None
        else:
            (s_hbm, pold_hbm, x_hbm, p_hbm, ip_hbm, *sc) = refs
        (srA, srB, s0, xA, xB, pA, pB, ipA, ipB, ka, kb, ibs,
         h0, h1, h2, h3, inA, inB, outA, outB) = sc
        hists = (h0, h1, h2, h3)
        srow = [srA, srB]
        xb = [xA, xB]
        pb = [pA, pB]
        ipb = [ipA, ipB]
        insem = [inA, inB]
        outsem = [outA, outB]
        wid = lax.axis_index("s") * 2 + lax.axis_index("c")
        pltpu.sync_copy(s_hbm.at[0], s0)

        def zero_x(j, c):
            zv = jnp.zeros((L,), jnp.float32)
            xA[pl.ds(j * L, L)] = zv
            xB[pl.ds(j * L, L)] = zv
            return c
        lax.fori_loop(0, DP // L, zero_x, 0)

        def issue_in(b, row):
            pltpu.async_copy(s_hbm.at[row], srow[b], insem[b])
            if not full_sort:
                @pl.when(row > 0)
                def _():
                    pltpu.async_copy(pold_hbm.at[row], pb[b], insem[b])

        def wait_in(b, row):
            pltpu.make_async_copy(s_hbm.at[row], srow[b], insem[b]).wait()
            if not full_sort:
                @pl.when(row > 0)
                def _():
                    pltpu.make_async_copy(
                        pold_hbm.at[row], pb[b], insem[b]).wait()

        def issue_out(b, row):
            pltpu.async_copy(xb[b], x_hbm.at[row], outsem[b])
            if full_sort:
                pltpu.async_copy(pb[b], p_hbm.at[row], outsem[b])
                pltpu.async_copy(ipb[b], ip_hbm.at[row], outsem[b])

        def wait_out(b, row):
            pltpu.make_async_copy(xb[b], x_hbm.at[row], outsem[b]).wait()
            if full_sort:
                pltpu.make_async_copy(pb[b], p_hbm.at[row], outsem[b]).wait()
                pltpu.make_async_copy(
                    ipb[b], ip_hbm.at[row], outsem[b]).wait()

        issue_in(0, wid)
        issue_in(1, NW + wid)

        def g_body(g, c):
            for b in range(2):
                row = (2 * g + b) * NW + wid
                wait_in(b, row)

                @pl.when(g > 0)
                def _(b=b, row=row):
                    wait_out(b, row)

                if full_sort:
                    @pl.when(row < N)
                    def _(b=b):
                        _sort_row(srow[b], ka, pb[b], kb, ibs, hists)
                else:
                    @pl.when(row == 0)
                    def _(b=b):
                        _sort_row(srow[b], ka, pb[b], kb, ibs, hists)

                def post_body(j, c2, b=b):
                    for u in range(4):
                        off = j * (L * 4) + u * L
                        mv = lax.iota(jnp.int32, L) + off
                        pv = pb[b][pl.ds(off, L)]
                        if full_sort:
                            plsc.store_scatter(ipb[b], [pv], mv)
                        sv = plsc.load_gather(srow[b], [pv + 1])
                        plsc.store_scatter(xb[b], [mv + 1], sv)
                        sec = plsc.load_gather(s0, [pv + 1])
                        plsc.store_scatter(xb[b], [mv + 1 + K], sec)
                    return c2
                @pl.when(row < N)
                def _(b=b, row=row):
                    lax.fori_loop(0, NV // 4, post_body, 0)
                    lane0 = lax.iota(jnp.int32, L) == 0
                    c0 = plsc.load_gather(s0,
                                          [lax.full((L,), row, jnp.int32)])
                    plsc.store_scatter(xb[b], [jnp.zeros((L,), jnp.int32)],
                                       c0, mask=lane0)

                if not full_sort:
                    @pl.when(row == 0)
                    def _(b=b):
                        def ip_body(j, c3):
                            for u in range(4):
                                off = j * (L * 4) + u * L
                                mv = lax.iota(jnp.int32, L) + off
                                pv = pb[b][pl.ds(off, L)]
                                plsc.store_scatter(ipb[b], [pv], mv)
                            return c3
                        lax.fori_loop(0, NV // 4, ip_body, 0)
                        pltpu.sync_copy(pb[b], p_hbm.at[0])
                        pltpu.sync_copy(ipb[b], ip_hbm.at[0])

                issue_out(b, row)

                @pl.when(g < RPW // 2 - 1)
                def _(b=b, row=row):
                    issue_in(b, row + 2 * NW)
            return c
        lax.fori_loop(0, RPW // 2, g_body, 0)
        for b in range(2):
            wait_out(b, (RPW - 2 + b) * NW + wid)

    return functools.partial(
        pl.kernel, out_type=out_type, mesh=_MESH, scratch_types=scratch,
        compiler_params=_SC_PARAMS)(body)


_sc_build_full = _make_sc_build(True)
_sc_build_reuse = _make_sc_build(False)


def _make_sc_bwd(qmode, split_ip):
    scratch = [
        pltpu.VMEM((DP,), jnp.float32),
        pltpu.VMEM((DP,), jnp.float32),
        pltpu.VMEM((K,), jnp.int32),
        pltpu.VMEM((K,), jnp.int32),
        pltpu.VMEM((NP,), jnp.float32),
        pltpu.VMEM((NP,), jnp.float32),
        pltpu.VMEM((NP,), jnp.float32),
        pltpu.SemaphoreType.DMA,
        pltpu.SemaphoreType.DMA,
        pltpu.SemaphoreType.DMA,
        pltpu.SemaphoreType.DMA,
    ]
    if qmode:
        out_type = (jax.ShapeDtypeStruct((1, NP), jnp.float32),
                    jax.ShapeDtypeStruct((NW, NP), jnp.float32))
    else:
        out_type = (jax.ShapeDtypeStruct((NP, NP), jnp.float32),
                    jax.ShapeDtypeStruct((NW, NP), jnp.float32))

    def body(*refs):
        if split_ip:
            (gl_hbm, ip_hbm, ip0_hbm, b_hbm, part_hbm, *sc) = refs
        else:
            (gl_hbm, ip_hbm, b_hbm, part_hbm, *sc) = refs
            ip0_hbm = None
        (glA, glB, ipbA, ipbB, brA, brB, rsum,
         inA, inB, outA, outB) = sc
        glrow = [glA, glB]
        ipbuf = [ipbA, ipbB]
        brow = [brA, brB]
        insem = [inA, inB]
        outsem = [outA, outB]
        wid = lax.axis_index("s") * 2 + lax.axis_index("c")

        def zero_body(j, c):
            zv = jnp.zeros((L,), jnp.float32)
            brA[pl.ds(j * L, L)] = zv
            brB[pl.ds(j * L, L)] = zv
            rsum[pl.ds(j * L, L)] = zv
            return c
        lax.fori_loop(0, NP // L, zero_body, 0)

        def issue_in(b, row):
            pltpu.async_copy(gl_hbm.at[row], glrow[b], insem[b])
            if split_ip:
                @pl.when(row == 0)
                def _():
                    pltpu.async_copy(ip0_hbm.at[0], ipbuf[b], insem[b])

                @pl.when(row > 0)
                def _():
                    pltpu.async_copy(ip_hbm.at[row], ipbuf[b], insem[b])
            else:
                pltpu.async_copy(ip_hbm.at[row], ipbuf[b], insem[b])

        def wait_in(b, row):
            pltpu.make_async_copy(gl_hbm.at[row], glrow[b], insem[b]).wait()
            pltpu.make_async_copy(ip_hbm.at[row], ipbuf[b], insem[b]).wait()

        issue_in(0, wid)
        issue_in(1, NW + wid)

        def g_body(g, c):
            for b in range(2):
                row = (2 * g + b) * NW + wid
                wait_in(b, row)
                if not qmode:
                    @pl.when(g > 0)
                    def _(b=b, row=row):
                        pltpu.make_async_copy(
                            brow[b], b_hbm.at[row], outsem[b]).wait()

                def gather_c(j, c2, b=b):
                    for u in range(_UR):
                        off = j * (L * _UR) + u * L
                        mv = lax.iota(jnp.int32, L) + off
                        ipv = ipbuf[b][pl.ds(off, L)]
                        cv = plsc.load_gather(glrow[b], [ipv + 1 + K])
                        old = plsc.load_gather(rsum, [mv + 1])
                        plsc.store_scatter(rsum, [mv + 1], old + cv)
                    return c2

                @pl.when(row < N)
                def _(b=b, row=row):
                    lax.fori_loop(0, NV // _UR, gather_c, 0)

                def gather_a(j, c2, b=b):
                    for u in range(_UR):
                        off = j * (L * _UR) + u * L
                        mv = lax.iota(jnp.int32, L) + off
                        ipv = ipbuf[b][pl.ds(off, L)]
                        av = plsc.load_gather(glrow[b], [ipv + 1])
                        plsc.store_scatter(brow[b], [mv + 1], av)
                    return c2

                if qmode:
                    @pl.when(row == 0)
                    def _(b=b):
                        lax.fori_loop(0, NV // _UR, gather_a, 0)
                        pltpu.sync_copy(brow[b], b_hbm.at[0])
                else:
                    @pl.when(row < N)
                    def _(b=b):
                        lax.fori_loop(0, NV // _UR, gather_a, 0)
                    pltpu.async_copy(brow[b], b_hbm.at[row], outsem[b])

                @pl.when(g < RPW // 2 - 1)
                def _(b=b, row=row):
                    issue_in(b, row + 2 * NW)
            return c
        lax.fori_loop(0, RPW // 2, g_body, 0)
        if not qmode:
            for b in range(2):
                pltpu.make_async_copy(
                    brow[b], b_hbm.at[(RPW - 2 + b) * NW + wid],
                    outsem[b]).wait()
        pltpu.sync_copy(rsum, part_hbm.at[wid])

    return functools.partial(
        pl.kernel, out_type=out_type, mesh=_MESH, scratch_types=scratch,
        compiler_params=_SC_PARAMS)(body)


_sc_bwd_q = _make_sc_bwd(True, False)
_sc_bwd_d = _make_sc_bwd(False, True)


def _bwd_d_body(b_ref, part_ref, glb_ref, f_ref, n_ref, feat_ref, out_ref):
    rowsum = jnp.sum(part_ref[...], axis=0, keepdims=True)
    glcol0 = glb_ref[...][:, 0:1]
    B = b_ref[...]
    ri = lax.broadcasted_iota(jnp.int32, (NP, NP), 0)
    ci = lax.broadcasted_iota(jnp.int32, (NP, NP), 1)
    M = B + jnp.where(ci == 0, glcol0, 0.0) + jnp.where(ri == 0, rowsum, 0.0)
    Mb = M.astype(jnp.bfloat16)
    Fb = f_ref[...].astype(jnp.bfloat16)
    dF = lax.dot_general(Mb, Fb, (((1,), (0,)), ((), ())),
                         preferred_element_type=jnp.float32)
    dF += lax.dot_general(Mb, Fb, (((0,), (0,)), ((), ())),
                          preferred_element_type=jnp.float32)
    F = f_ref[...]
    n = n_ref[...]
    dot = jnp.sum(dF * F, axis=1, keepdims=True)
    g = (dF - jnp.where(n > EPS_N, dot * F, 0.0)) / jnp.maximum(n, EPS_N)
    rif = lax.broadcasted_iota(jnp.int32, (NP, FEAT), 0)
    upd = (rif >= 1) & (rif < N)
    out_ref[...] = feat_ref[...] - LR * jnp.where(upd, g, 0.0)


def _bwd_d(Bmat, partials, glb, F, n, feat_p):
    return pl.pallas_call(
        _bwd_d_body,
        out_shape=jax.ShapeDtypeStruct((NP, FEAT), jnp.float32),
    )(Bmat, partials, glb, F, n, feat_p)


def _bwd_q_body(a_ref, part_ref, glb_ref, f_ref, n_ref, feat_ref, out_ref):
    rowsum = jnp.sum(part_ref[...], axis=0, keepdims=True)
    t = a_ref[...] + rowsum
    F = f_ref[...]
    glcol0 = glb_ref[...][:, 0:1]
    dF0 = lax.dot_general(t, F, (((1,), (0,)), ((), ())),
                          preferred_element_type=jnp.float32)
    dF0 += lax.dot_general(glcol0, F, (((0,), (0,)), ((), ())),
                           preferred_element_type=jnp.float32)
    dF0 += glb_ref[...][0:1, 0:1] * F[0:1]
    F0 = F[0:1]
    n0 = n_ref[...][0:1]
    dot = jnp.sum(dF0 * F0, axis=1, keepdims=True)
    g0 = (dF0 - jnp.where(n0 > EPS_N, dot * F0, 0.0)) / jnp.maximum(n0, EPS_N)
    out_ref[...] = feat_ref[...]
    out_ref[0:1, :] = feat_ref[0:1, :] - LR * g0


def _bwd_q(arow0, partials, glb, F, n, feat_p):
    return pl.pallas_call(
        _bwd_q_body,
        out_shape=jax.ShapeDtypeStruct((NP, FEAT), jnp.float32),
    )(arow0, partials, glb, F, n, feat_p)


def _prep_params(p):
    scal = jnp.stack([p['in1_w'], p['in1_b'], p['in2_w'], p['in2_b'],
                      p['in3_w'], p['in3_b']])
    w1 = jnp.pad(p['l1_W'], ((0, 0), (0, DP - DIN))).astype(jnp.bfloat16)
    w2 = p['l2_W'].astype(jnp.bfloat16)
    w3 = jnp.pad(p['l3_W'], ((0, DP - DIN), (0, 0))).astype(jnp.bfloat16)
    b1 = p['l1_b'][None, :]
    b2 = p['l2_b'][None, :]
    b3 = jnp.pad(p['l3_b'], (0, DP - DIN))[None, :]
    return dict(scal=scal, w1=w1, b1=b1, w2=w2, b2=b2, w3=w3, b3=b3)


def _stage_q(feat_p, wp):
    S, F, n = _similarity(feat_p)
    x_p, p, ip = _sc_build_full(S)
    gl = _dni(x_p, wp)
    glb = lax.slice(gl, (0, 0), (NP, 128))
    arow0, partials = _sc_bwd_q(gl, ip)
    feat_p = _bwd_q(arow0, partials, glb, F, n, feat_p)
    return feat_p, p, ip


def _stage_d(feat_p, wp, p_old, ip_old):
    S, F, n = _similarity(feat_p)
    x_p, p0, ip0 = _sc_build_reuse(S, p_old)
    gl = _dni(x_p, wp)
    glb = lax.slice(gl, (0, 0), (NP, 128))
    Bmat, partials = _sc_bwd_d(gl, ip_old, ip0)
    feat_p = _bwd_d(Bmat, partials, glb, F, n, feat_p)
    return feat_p


def kernel(feat_query, feat_database, params):
    feat = jnp.concatenate([feat_query, feat_database], axis=0)
    feat_p = jnp.pad(feat, ((0, NP - N), (0, 0)))
    wq = _prep_params(params['query'])
    wd = _prep_params(params['database'])
    feat_p, p, ip = _stage_q(feat_p, wq)
    feat_p = _stage_d(feat_p, wd, p, ip)
    feat_p, p, ip = _stage_q(feat_p, wq)
    feat_p = _stage_d(feat_p, wd, p, ip)
    out = _final_row(feat_p)
    return out[0, 1:N]


# speedup vs baseline: 146.9102x; 1.0265x over previous
import functools

import jax
import jax.numpy as jnp
from jax import lax
from jax.experimental import pallas as pl
from jax.experimental.pallas import tpu as pltpu
from jax.experimental.pallas import tpu_sc as plsc

FEAT = 512
K = 1024
N = K + 1
NP = 1152
DIN = 2 * K + 1
DP = 2176
HID = 2048
LR = 1e-3
EPS_N = 1e-12
EPS_IN = 1e-5

L = 16
NV = K // L
NB = 256
NW = 32
RPW = NP // NW

_MESH = plsc.VectorSubcoreMesh(core_axis_name="c", subcore_axis_name="s")
_SC_PARAMS = pltpu.CompilerParams(needs_layout_passes=False)


def _sim_body(feat_ref, s_ref, f_ref, n_ref):
    x = feat_ref[...]
    n = jnp.sqrt(jnp.sum(x * x, axis=1, keepdims=True))
    F = x / jnp.maximum(n, EPS_N)
    s_ref[...] = lax.dot_general(
        F, F, (((1,), (1,)), ((), ())), preferred_element_type=jnp.float32)
    f_ref[...] = F
    n_ref[...] = n


def _similarity(feat_p):
    return pl.pallas_call(
        _sim_body,
        out_shape=(
            jax.ShapeDtypeStruct((NP, NP), jnp.float32),
            jax.ShapeDtypeStruct((NP, FEAT), jnp.float32),
            jax.ShapeDtypeStruct((NP, 1), jnp.float32),
        ),
    )(feat_p)


def _final_body(feat_ref, out_ref):
    x = feat_ref[...]
    n = jnp.sqrt(jnp.sum(x * x, axis=1, keepdims=True))
    F = x / jnp.maximum(n, EPS_N)
    out_ref[...] = lax.dot_general(
        F[0:1], F, (((1,), (1,)), ((), ())),
        preferred_element_type=jnp.float32)


def _final_row(feat_p):
    return pl.pallas_call(
        _final_body,
        out_shape=jax.ShapeDtypeStruct((1, NP), jnp.float32),
    )(feat_p)


def _dni_body(scal_ref, x_ref, w1_ref, b1_ref, w2_ref, b2_ref, w3_ref,
              b3_ref, out_ref):
    x = x_ref[...]
    mask = lax.broadcasted_iota(jnp.int32, x.shape, 1) < DIN

    def inorm(h, d, w, b, msk=None):
        mean = jnp.sum(h, axis=1, keepdims=True) / d
        var = jnp.sum(h * h, axis=1, keepdims=True) / d - mean * mean
        out = (h - mean) * lax.rsqrt(var + EPS_IN) * w + b
        if msk is not None:
            out = jnp.where(msk, out, 0.0)
        return out

    def mm(a, w_ref):
        return lax.dot_general(
            a.astype(jnp.bfloat16), w_ref[...],
            (((1,), (1,)), ((), ())), preferred_element_type=jnp.float32)

    h = inorm(x, DIN, scal_ref[0], scal_ref[1], mask)
    h = jnp.maximum(mm(h, w1_ref) + b1_ref[...], 0.0)
    h = inorm(h, HID, scal_ref[2], scal_ref[3])
    h = jnp.maximum(mm(h, w2_ref) + b2_ref[...], 0.0)
    h = inorm(h, HID, scal_ref[4], scal_ref[5])
    out_ref[...] = mm(h, w3_ref) + b3_ref[...]


def _dni(x_p, wp):
    br = 128
    w_spec = lambda shape: pl.BlockSpec(shape, lambda i: (0, 0))
    return pl.pallas_call(
        _dni_body,
        grid=(NP // br,),
        in_specs=[
            pl.BlockSpec(memory_space=pltpu.SMEM),
            pl.BlockSpec((br, DP), lambda i: (i, 0)),
            w_spec((HID, DP)), w_spec((1, HID)),
            w_spec((HID, HID)), w_spec((1, HID)),
            w_spec((DP, HID)), w_spec((1, DP)),
        ],
        out_specs=pl.BlockSpec((br, DP), lambda i: (i, 0)),
        out_shape=jax.ShapeDtypeStruct((NP, DP), jnp.float32),
    )(wp['scal'], x_p, wp['w1'], wp['b1'], wp['w2'], wp['b2'],
      wp['w3'], wp['b3'])


def _monotone_desc(u):
    m = lax.shift_right_arithmetic(u, 31) | jnp.int32(-2 ** 31)
    return ~(u ^ m)


def _digit(k, shift):
    return lax.shift_right_logical(k, shift) & 0xFF


_SHIFTS = (0, 8, 16, 24)
_UR = 8


def _sort_row(srow, ka, ia, kb, ib, hists):
    zv = jnp.zeros((L,), jnp.int32)
    for h in hists:
        for b in range(NB // L):
            h[pl.ds(b * L, L)] = zv

    ones = jnp.full((L,), 1, jnp.int32)

    def lh_body(j, c):
        for u in range(_UR):
            off = j * (L * _UR) + u * L
            kv = plsc.load_gather(srow, [lax.iota(jnp.int32, L) + (off + 1)])
            m = _monotone_desc(plsc.bitcast(kv, jnp.int32))
            ka[pl.ds(off, L)] = m
            for pn, shift in enumerate(_SHIFTS):
                plsc.addupdate_scatter(hists[pn], [_digit(m, shift)], ones)
        return c
    lax.fori_loop(0, NV // _UR, lh_body, 0)

    bufs = [(ka, ia), (kb, ib)]
    for pn, shift in enumerate(_SHIFTS):
        hist = hists[pn]
        carry = 0
        for b in range(NB // L):
            v = hist[pl.ds(b * L, L)]
            cs = plsc.cumsum(v)
            hist[pl.ds(b * L, L)] = cs - v + carry
            carry = carry + jnp.sum(v)
        src_k, src_i = bufs[pn % 2]
        dst_k, dst_i = bufs[(pn + 1) % 2]

        def perm_body(j, c, pn=pn, shift=shift, hist=hist, src_k=src_k,
                      src_i=src_i, dst_k=dst_k, dst_i=dst_i):
            for u in range(_UR):
                off = j * (L * _UR) + u * L
                k = src_k[pl.ds(off, L)]
                if pn == 0:
                    iv = lax.iota(jnp.int32, L) + off
                else:
                    iv = src_i[pl.ds(off, L)]
                d = _digit(k, shift)
                base = plsc.load_gather(hist, [d])
                occ, lastm = plsc.scan_count(d)
                pos = base + occ - 1
                plsc.store_scatter(dst_k, [pos], k)
                plsc.store_scatter(dst_i, [pos], iv)
                plsc.addupdate_scatter(hist, [d], occ, mask=lastm)
            return c
        lax.fori_loop(0, NV // _UR, perm_body, 0)


def _make_sc_build(full_sort):
    scratch = [
        pltpu.VMEM((NP,), jnp.float32),
        pltpu.VMEM((NP,), jnp.float32),
        pltpu.VMEM((NP,), jnp.float32),
        pltpu.VMEM((DP,), jnp.float32),
        pltpu.VMEM((DP,), jnp.float32),
        pltpu.VMEM((K,), jnp.int32),
        pltpu.VMEM((K,), jnp.int32),
        pltpu.VMEM((K,), jnp.int32),
        pltpu.VMEM((K,), jnp.int32),
        pltpu.VMEM((K,), jnp.int32),
        pltpu.VMEM((K,), jnp.int32),
        pltpu.VMEM((K,), jnp.int32),
        pltpu.VMEM((NB,), jnp.int32),
        pltpu.VMEM((NB,), jnp.int32),
        pltpu.VMEM((NB,), jnp.int32),
        pltpu.VMEM((NB,), jnp.int32),
        pltpu.SemaphoreType.DMA,
        pltpu.SemaphoreType.DMA,
        pltpu.SemaphoreType.DMA,
        pltpu.SemaphoreType.DMA,
    ]
    if full_sort:
        out_type = (
            jax.ShapeDtypeStruct((NP, DP), jnp.float32),
            jax.ShapeDtypeStruct((NP, K), jnp.int32),
            jax.ShapeDtypeStruct((NP, K), jnp.int32),
        )
    else:
        out_type = (
            jax.ShapeDtypeStruct((NP, DP), jnp.float32),
            jax.ShapeDtypeStruct((1, K), jnp.int32),
            jax.ShapeDtypeStruct((1, K), jnp.int32),
        )

    def body(*refs):
        if full_sort:
            (s_hbm, x_hbm, p_hbm, ip_hbm, *sc) = refs
            pold_hbm = None
        else:
            (s_hbm, pold_hbm, x_hbm, p_hbm, ip_hbm, *sc) = refs
        (srA, srB, s0, xA, xB, pA, pB, ipA, ipB, ka, kb, ibs,
         h0, h1, h2, h3, inA, inB, outA, outB) = sc
        hists = (h0, h1, h2, h3)
        srow = [srA, srB]
        xb = [xA, xB]
        pb = [pA, pB]
        ipb = [ipA, ipB]
        insem = [inA, inB]
        outsem = [outA, outB]
        wid = lax.axis_index("s") * 2 + lax.axis_index("c")
        pltpu.sync_copy(s_hbm.at[0], s0)

        def zero_x(j, c):
            zv = jnp.zeros((L,), jnp.float32)
            xA[pl.ds(j * L, L)] = zv
            xB[pl.ds(j * L, L)] = zv
            return c
        lax.fori_loop(0, DP // L, zero_x, 0)

        def issue_in(b, row):
            pltpu.async_copy(s_hbm.at[row], srow[b], insem[b])
            if not full_sort:
                @pl.when(row > 0)
                def _():
                    pltpu.async_copy(pold_hbm.at[row], pb[b], insem[b])

        def wait_in(b, row):
            pltpu.make_async_copy(s_hbm.at[row], srow[b], insem[b]).wait()
            if not full_sort:
                @pl.when(row > 0)
                def _():
                    pltpu.make_async_copy(
                        pold_hbm.at[row], pb[b], insem[b]).wait()

        def issue_out(b, row):
            pltpu.async_copy(xb[b], x_hbm.at[row], outsem[b])
            if full_sort:
                pltpu.async_copy(pb[b], p_hbm.at[row], outsem[b])
                pltpu.async_copy(ipb[b], ip_hbm.at[row], outsem[b])

        def wait_out(b, row):
            pltpu.make_async_copy(xb[b], x_hbm.at[row], outsem[b]).wait()
            if full_sort:
                pltpu.make_async_copy(pb[b], p_hbm.at[row], outsem[b]).wait()
                pltpu.make_async_copy(
                    ipb[b], ip_hbm.at[row], outsem[b]).wait()

        issue_in(0, wid)
        issue_in(1, NW + wid)

        def g_body(g, c):
            for b in range(2):
                row = (2 * g + b) * NW + wid
                wait_in(b, row)

                @pl.when(g > 0)
                def _(b=b, row=row):
                    wait_out(b, row)

                if full_sort:
                    @pl.when(row < N)
                    def _(b=b):
                        _sort_row(srow[b], ka, pb[b], kb, ibs, hists)
                else:
                    @pl.when(row == 0)
                    def _(b=b):
                        _sort_row(srow[b], ka, pb[b], kb, ibs, hists)

                def post_body(j, c2, b=b):
                    for u in range(4):
                        off = j * (L * 4) + u * L
                        mv = lax.iota(jnp.int32, L) + off
                        pv = pb[b][pl.ds(off, L)]
                        if full_sort:
                            plsc.store_scatter(ipb[b], [pv], mv)
                        sv = plsc.load_gather(srow[b], [pv + 1])
                        plsc.store_scatter(xb[b], [mv + 1], sv)
                        sec = plsc.load_gather(s0, [pv + 1])
                        plsc.store_scatter(xb[b], [mv + 1 + K], sec)
                    return c2
                @pl.when(row < N)
                def _(b=b, row=row):
                    lax.fori_loop(0, NV // 4, post_body, 0)
                    lane0 = lax.iota(jnp.int32, L) == 0
                    c0 = plsc.load_gather(s0,
                                          [lax.full((L,), row, jnp.int32)])
                    plsc.store_scatter(xb[b], [jnp.zeros((L,), jnp.int32)],
                                       c0, mask=lane0)

                if not full_sort:
                    @pl.when(row == 0)
                    def _(b=b):
                        def ip_body(j, c3):
                            for u in range(4):
                                off = j * (L * 4) + u * L
                                mv = lax.iota(jnp.int32, L) + off
                                pv = pb[b][pl.ds(off, L)]
                                plsc.store_scatter(ipb[b], [pv], mv)
                            return c3
                        lax.fori_loop(0, NV // 4, ip_body, 0)
                        pltpu.sync_copy(pb[b], p_hbm.at[0])
                        pltpu.sync_copy(ipb[b], ip_hbm.at[0])

                issue_out(b, row)

                @pl.when(g < RPW // 2 - 1)
                def _(b=b, row=row):
                    issue_in(b, row + 2 * NW)
            return c
        lax.fori_loop(0, RPW // 2, g_body, 0)
        for b in range(2):
            wait_out(b, (RPW - 2 + b) * NW + wid)

    return functools.partial(
        pl.kernel, out_type=out_type, mesh=_MESH, scratch_types=scratch,
        compiler_params=_SC_PARAMS)(body)


_sc_build_full = _make_sc_build(True)
_sc_build_reuse = _make_sc_build(False)


def _make_sc_bwd(qmode, split_ip):
    scratch = [
        pltpu.VMEM((DP,), jnp.float32),
        pltpu.VMEM((DP,), jnp.float32),
        pltpu.VMEM((K,), jnp.int32),
        pltpu.VMEM((K,), jnp.int32),
        pltpu.VMEM((NP,), jnp.float32),
        pltpu.VMEM((NP,), jnp.float32),
        pltpu.VMEM((NP,), jnp.float32),
        pltpu.SemaphoreType.DMA,
        pltpu.SemaphoreType.DMA,
        pltpu.SemaphoreType.DMA,
        pltpu.SemaphoreType.DMA,
    ]
    if qmode:
        out_type = (jax.ShapeDtypeStruct((1, NP), jnp.float32),
                    jax.ShapeDtypeStruct((NW, NP), jnp.float32))
    else:
        out_type = (jax.ShapeDtypeStruct((NP, NP), jnp.float32),
                    jax.ShapeDtypeStruct((NW, NP), jnp.float32))

    def body(*refs):
        if split_ip:
            (gl_hbm, ip_hbm, ip0_hbm, b_hbm, part_hbm, *sc) = refs
        else:
            (gl_hbm, ip_hbm, b_hbm, part_hbm, *sc) = refs
            ip0_hbm = None
        (glA, glB, ipbA, ipbB, brA, brB, rsum,
         inA, inB, outA, outB) = sc
        glrow = [glA, glB]
        ipbuf = [ipbA, ipbB]
        brow = [brA, brB]
        insem = [inA, inB]
        outsem = [outA, outB]
        wid = lax.axis_index("s") * 2 + lax.axis_index("c")

        def zero_body(j, c):
            zv = jnp.zeros((L,), jnp.float32)
            brA[pl.ds(j * L, L)] = zv
            brB[pl.ds(j * L, L)] = zv
            rsum[pl.ds(j * L, L)] = zv
            return c
        lax.fori_loop(0, NP // L, zero_body, 0)

        def issue_in(b, row):
            pltpu.async_copy(gl_hbm.at[row], glrow[b], insem[b])
            if split_ip:
                @pl.when(row == 0)
                def _():
                    pltpu.async_copy(ip0_hbm.at[0], ipbuf[b], insem[b])

                @pl.when(row > 0)
                def _():
                    pltpu.async_copy(ip_hbm.at[row], ipbuf[b], insem[b])
            else:
                pltpu.async_copy(ip_hbm.at[row], ipbuf[b], insem[b])

        def wait_in(b, row):
            pltpu.make_async_copy(gl_hbm.at[row], glrow[b], insem[b]).wait()
            pltpu.make_async_copy(ip_hbm.at[row], ipbuf[b], insem[b]).wait()

        issue_in(0, wid)
        issue_in(1, NW + wid)

        def g_body(g, c):
            for b in range(2):
                row = (2 * g + b) * NW + wid
                wait_in(b, row)
                if not qmode:
                    @pl.when(g > 0)
                    def _(b=b, row=row):
                        pltpu.make_async_copy(
                            brow[b], b_hbm.at[row], outsem[b]).wait()

                def gather_c(j, c2, b=b):
                    for u in range(_UR):
                        off = j * (L * _UR) + u * L
                        mv = lax.iota(jnp.int32, L) + off
                        ipv = ipbuf[b][pl.ds(off, L)]
                        cv = plsc.load_gather(glrow[b], [ipv + 1 + K])
                        old = plsc.load_gather(rsum, [mv + 1])
                        plsc.store_scatter(rsum, [mv + 1], old + cv)
                    return c2

                @pl.when(row < N)
                def _(b=b, row=row):
                    lax.fori_loop(0, NV // _UR, gather_c, 0)

                def gather_a(j, c2, b=b):
                    for u in range(_UR):
                        off = j * (L * _UR) + u * L
                        mv = lax.iota(jnp.int32, L) + off
                        ipv = ipbuf[b][pl.ds(off, L)]
                        av = plsc.load_gather(glrow[b], [ipv + 1])
                        plsc.store_scatter(brow[b], [mv + 1], av)
                    return c2

                if qmode:
                    @pl.when(row == 0)
                    def _(b=b):
                        lax.fori_loop(0, NV // _UR, gather_a, 0)
                        pltpu.sync_copy(brow[b], b_hbm.at[0])
                else:
                    @pl.when(row < N)
                    def _(b=b):
                        lax.fori_loop(0, NV // _UR, gather_a, 0)
                    pltpu.async_copy(brow[b], b_hbm.at[row], outsem[b])

                @pl.when(g < RPW // 2 - 1)
                def _(b=b, row=row):
                    issue_in(b, row + 2 * NW)
            return c
        lax.fori_loop(0, RPW // 2, g_body, 0)
        if not qmode:
            for b in range(2):
                pltpu.make_async_copy(
                    brow[b], b_hbm.at[(RPW - 2 + b) * NW + wid],
                    outsem[b]).wait()
        pltpu.sync_copy(rsum, part_hbm.at[wid])

    return functools.partial(
        pl.kernel, out_type=out_type, mesh=_MESH, scratch_types=scratch,
        compiler_params=_SC_PARAMS)(body)


_sc_bwd_q = _make_sc_bwd(True, False)
_sc_bwd_d = _make_sc_bwd(False, True)


def _bwd_d_body(b_ref, part_ref, glb_ref, f_ref, n_ref, feat_ref, out_ref):
    rowsum = jnp.sum(part_ref[...], axis=0, keepdims=True)
    glcol0 = glb_ref[...][:, 0:1]
    B = b_ref[...]
    ri = lax.broadcasted_iota(jnp.int32, (NP, NP), 0)
    ci = lax.broadcasted_iota(jnp.int32, (NP, NP), 1)
    M = B + jnp.where(ci == 0, glcol0, 0.0) + jnp.where(ri == 0, rowsum, 0.0)
    Mb = M.astype(jnp.bfloat16)
    Fb = f_ref[...].astype(jnp.bfloat16)
    dF = lax.dot_general(Mb, Fb, (((1,), (0,)), ((), ())),
                         preferred_element_type=jnp.float32)
    dF += lax.dot_general(Mb, Fb, (((0,), (0,)), ((), ())),
                          preferred_element_type=jnp.float32)
    F = f_ref[...]
    n = n_ref[...]
    dot = jnp.sum(dF * F, axis=1, keepdims=True)
    g = (dF - jnp.where(n > EPS_N, dot * F, 0.0)) / jnp.maximum(n, EPS_N)
    rif = lax.broadcasted_iota(jnp.int32, (NP, FEAT), 0)
    upd = (rif >= 1) & (rif < N)
    out_ref[...] = feat_ref[...] - LR * jnp.where(upd, g, 0.0)


def _bwd_d(Bmat, partials, glb, F, n, feat_p):
    return pl.pallas_call(
        _bwd_d_body,
        out_shape=jax.ShapeDtypeStruct((NP, FEAT), jnp.float32),
    )(Bmat, partials, glb, F, n, feat_p)


def _bwd_q_body(a_ref, part_ref, glb_ref, f_ref, n_ref, feat_ref, out_ref):
    rowsum = jnp.sum(part_ref[...], axis=0, keepdims=True)
    t = a_ref[...] + rowsum
    F = f_ref[...]
    glcol0 = glb_ref[...][:, 0:1]
    dF0 = lax.dot_general(t, F, (((1,), (0,)), ((), ())),
                          preferred_element_type=jnp.float32)
    dF0 += lax.dot_general(glcol0, F, (((0,), (0,)), ((), ())),
                           preferred_element_type=jnp.float32)
    dF0 += glb_ref[...][0:1, 0:1] * F[0:1]
    F0 = F[0:1]
    n0 = n_ref[...][0:1]
    dot = jnp.sum(dF0 * F0, axis=1, keepdims=True)
    g0 = (dF0 - jnp.where(n0 > EPS_N, dot * F0, 0.0)) / jnp.maximum(n0, EPS_N)
    out_ref[...] = feat_ref[...]
    out_ref[0:1, :] = feat_ref[0:1, :] - LR * g0


def _bwd_q(arow0, partials, glb, F, n, feat_p):
    return pl.pallas_call(
        _bwd_q_body,
        out_shape=jax.ShapeDtypeStruct((NP, FEAT), jnp.float32),
    )(arow0, partials, glb, F, n, feat_p)


def _prep_params(p):
    scal = jnp.stack([p['in1_w'], p['in1_b'], p['in2_w'], p['in2_b'],
                      p['in3_w'], p['in3_b']])
    w1 = jnp.pad(p['l1_W'], ((0, 0), (0, DP - DIN))).astype(jnp.bfloat16)
    w2 = p['l2_W'].astype(jnp.bfloat16)
    w3 = jnp.pad(p['l3_W'], ((0, DP - DIN), (0, 0))).astype(jnp.bfloat16)
    b1 = p['l1_b'][None, :]
    b2 = p['l2_b'][None, :]
    b3 = jnp.pad(p['l3_b'], (0, DP - DIN))[None, :]
    return dict(scal=scal, w1=w1, b1=b1, w2=w2, b2=b2, w3=w3, b3=b3)


def _stage_q(feat_p, wp):
    S, F, n = _similarity(feat_p)
    x_p, p, ip = _sc_build_full(S)
    gl = _dni(x_p, wp)
    glb = lax.slice(gl, (0, 0), (NP, 128))
    arow0, partials = _sc_bwd_q(gl, ip)
    feat_p = _bwd_q(arow0, partials, glb, F, n, feat_p)
    return feat_p, p, ip


def _stage_d(feat_p, wp, p_old, ip_old):
    S, F, n = _similarity(feat_p)
    x_p, p0, ip0 = _sc_build_reuse(S, p_old)
    gl = _dni(x_p, wp)
    glb = lax.slice(gl, (0, 0), (NP, 128))
    Bmat, partials = _sc_bwd_d(gl, ip_old, ip0)
    feat_p = _bwd_d(Bmat, partials, glb, F, n, feat_p)
    return feat_p


def kernel(feat_query, feat_database, params):
    feat = jnp.concatenate([feat_query, feat_database], axis=0)
    feat_p = jnp.pad(feat, ((0, NP - N), (0, 0)))
    wq = _prep_params(params['query'])
    wd = _prep_params(params['database'])
    feat_p, p, ip = _stage_q(feat_p, wq)
    feat_p = _stage_d(feat_p, wd, p, ip)
    feat_p, p, ip = _stage_q(feat_p, wq)
    feat_p = _stage_d(feat_p, wd, p, ip)
    out = _final_row(feat_p)
    return out[0, 1:N]


# speedup vs baseline: 148.7988x; 1.0129x over previous
import functools

import jax
import jax.numpy as jnp
from jax import lax
from jax.experimental import pallas as pl
from jax.experimental.pallas import tpu as pltpu
from jax.experimental.pallas import tpu_sc as plsc

FEAT = 512
K = 1024
N = K + 1
NP = 1152
DIN = 2 * K + 1
DP = 2176
HID = 2048
LR = 1e-3
EPS_N = 1e-12
EPS_IN = 1e-5

L = 16
NV = K // L
NB = 256
NW = 32
RPW = NP // NW

_MESH = plsc.VectorSubcoreMesh(core_axis_name="c", subcore_axis_name="s")
_SC_PARAMS = pltpu.CompilerParams(needs_layout_passes=False)


def _sim_body(feat_ref, s_ref, f_ref, n_ref):
    x = feat_ref[...]
    n = jnp.sqrt(jnp.sum(x * x, axis=1, keepdims=True))
    F = x / jnp.maximum(n, EPS_N)
    s_ref[...] = lax.dot_general(
        F, F, (((1,), (1,)), ((), ())), preferred_element_type=jnp.float32)
    f_ref[...] = F
    n_ref[...] = n


def _similarity(feat_p):
    return pl.pallas_call(
        _sim_body,
        out_shape=(
            jax.ShapeDtypeStruct((NP, NP), jnp.float32),
            jax.ShapeDtypeStruct((NP, FEAT), jnp.float32),
            jax.ShapeDtypeStruct((NP, 1), jnp.float32),
        ),
    )(feat_p)


def _final_body(feat_ref, out_ref):
    x = feat_ref[...]
    n = jnp.sqrt(jnp.sum(x * x, axis=1, keepdims=True))
    F = x / jnp.maximum(n, EPS_N)
    out_ref[...] = lax.dot_general(
        F[0:1], F, (((1,), (1,)), ((), ())),
        preferred_element_type=jnp.float32)


def _final_row(feat_p):
    return pl.pallas_call(
        _final_body,
        out_shape=jax.ShapeDtypeStruct((1, NP), jnp.float32),
    )(feat_p)


def _dni_body(scal_ref, x_ref, w1_ref, b1_ref, w2_ref, b2_ref, w3_ref,
              b3_ref, out_ref):
    x = x_ref[...]
    mask = lax.broadcasted_iota(jnp.int32, x.shape, 1) < DIN

    def inorm(h, d, w, b, msk=None):
        mean = jnp.sum(h, axis=1, keepdims=True) / d
        var = jnp.sum(h * h, axis=1, keepdims=True) / d - mean * mean
        out = (h - mean) * lax.rsqrt(var + EPS_IN) * w + b
        if msk is not None:
            out = jnp.where(msk, out, 0.0)
        return out

    def mm(a, w_ref):
        return lax.dot_general(
            a.astype(jnp.bfloat16), w_ref[...],
            (((1,), (1,)), ((), ())), preferred_element_type=jnp.float32)

    h = inorm(x, DIN, scal_ref[0], scal_ref[1], mask)
    h = jnp.maximum(mm(h, w1_ref) + b1_ref[...], 0.0)
    h = inorm(h, HID, scal_ref[2], scal_ref[3])
    h = jnp.maximum(mm(h, w2_ref) + b2_ref[...], 0.0)
    h = inorm(h, HID, scal_ref[4], scal_ref[5])
    out_ref[...] = mm(h, w3_ref) + b3_ref[...]


def _dni(x_p, wp):
    br = 128
    w_spec = lambda shape: pl.BlockSpec(shape, lambda i: (0, 0))
    return pl.pallas_call(
        _dni_body,
        grid=(NP // br,),
        in_specs=[
            pl.BlockSpec(memory_space=pltpu.SMEM),
            pl.BlockSpec((br, DP), lambda i: (i, 0)),
            w_spec((HID, DP)), w_spec((1, HID)),
            w_spec((HID, HID)), w_spec((1, HID)),
            w_spec((DP, HID)), w_spec((1, DP)),
        ],
        out_specs=pl.BlockSpec((br, DP), lambda i: (i, 0)),
        out_shape=jax.ShapeDtypeStruct((NP, DP), jnp.float32),
    )(wp['scal'], x_p, wp['w1'], wp['b1'], wp['w2'], wp['b2'],
      wp['w3'], wp['b3'])


def _monotone_desc(u):
    m = lax.shift_right_arithmetic(u, 31) | jnp.int32(-2 ** 31)
    return ~(u ^ m)


def _digit(k, shift):
    return lax.shift_right_logical(k, shift) & 0xFF


_SHIFTS = (0, 8, 16, 24)
_UR = 8


def _sort_row(srow, ka, ia, kb, ib, hists):
    zv = jnp.zeros((L,), jnp.int32)
    for h in hists:
        for b in range(NB // L):
            h[pl.ds(b * L, L)] = zv

    ones = jnp.full((L,), 1, jnp.int32)

    def lh_body(j, c):
        for u in range(_UR):
            off = j * (L * _UR) + u * L
            kv = plsc.load_gather(srow, [lax.iota(jnp.int32, L) + (off + 1)])
            m = _monotone_desc(plsc.bitcast(kv, jnp.int32))
            ka[pl.ds(off, L)] = m
            for pn, shift in enumerate(_SHIFTS):
                plsc.addupdate_scatter(hists[pn], [_digit(m, shift)], ones)
        return c
    lax.fori_loop(0, NV // _UR, lh_body, 0)

    bufs = [(ka, ia), (kb, ib)]
    for pn, shift in enumerate(_SHIFTS):
        hist = hists[pn]
        carry = 0
        for b in range(NB // L):
            v = hist[pl.ds(b * L, L)]
            cs = plsc.cumsum(v)
            hist[pl.ds(b * L, L)] = cs - v + carry
            carry = carry + jnp.sum(v)
        src_k, src_i = bufs[pn % 2]
        dst_k, dst_i = bufs[(pn + 1) % 2]

        def perm_body(j, c, pn=pn, shift=shift, hist=hist, src_k=src_k,
                      src_i=src_i, dst_k=dst_k, dst_i=dst_i):
            for u in range(_UR):
                off = j * (L * _UR) + u * L
                k = src_k[pl.ds(off, L)]
                if pn == 0:
                    iv = lax.iota(jnp.int32, L) + off
                else:
                    iv = src_i[pl.ds(off, L)]
                d = _digit(k, shift)
                base = plsc.load_gather(hist, [d])
                occ, lastm = plsc.scan_count(d)
                pos = base + occ - 1
                plsc.store_scatter(dst_k, [pos], k)
                plsc.store_scatter(dst_i, [pos], iv)
                plsc.addupdate_scatter(hist, [d], occ, mask=lastm)
            return c
        lax.fori_loop(0, NV // _UR, perm_body, 0)


def _make_sc_build(full_sort):
    scratch = [
        pltpu.VMEM((NP,), jnp.float32),
        pltpu.VMEM((NP,), jnp.float32),
        pltpu.VMEM((NP,), jnp.float32),
        pltpu.VMEM((DP,), jnp.float32),
        pltpu.VMEM((DP,), jnp.float32),
        pltpu.VMEM((K,), jnp.int32),
        pltpu.VMEM((K,), jnp.int32),
        pltpu.VMEM((K,), jnp.int32),
        pltpu.VMEM((K,), jnp.int32),
        pltpu.VMEM((K,), jnp.int32),
        pltpu.VMEM((K,), jnp.int32),
        pltpu.VMEM((K,), jnp.int32),
        pltpu.VMEM((NB,), jnp.int32),
        pltpu.VMEM((NB,), jnp.int32),
        pltpu.VMEM((NB,), jnp.int32),
        pltpu.VMEM((NB,), jnp.int32),
        pltpu.SemaphoreType.DMA,
        pltpu.SemaphoreType.DMA,
        pltpu.SemaphoreType.DMA,
        pltpu.SemaphoreType.DMA,
    ]
    if full_sort:
        out_type = (
            jax.ShapeDtypeStruct((NP, DP), jnp.float32),
            jax.ShapeDtypeStruct((NP, K), jnp.int32),
            jax.ShapeDtypeStruct((NP, K), jnp.int32),
        )
    else:
        out_type = (
            jax.ShapeDtypeStruct((NP, DP), jnp.float32),
            jax.ShapeDtypeStruct((1, K), jnp.int32),
            jax.ShapeDtypeStruct((1, K), jnp.int32),
        )

    def body(*refs):
        if full_sort:
            (s_hbm, x_hbm, p_hbm, ip_hbm, *sc) = refs
            pold_hbm = None
        else:
            (s_hbm, s0_hbm, pold_hbm, x_hbm, p_hbm, ip_hbm, *sc) = refs
        (srA, srB, s0, xA, xB, pA, pB, ipA, ipB, ka, kb, ibs,
         h0, h1, h2, h3, inA, inB, outA, outB) = sc
        hists = (h0, h1, h2, h3)
        srow = [srA, srB]
        xb = [xA, xB]
        pb = [pA, pB]
        ipb = [ipA, ipB]
        insem = [inA, inB]
        outsem = [outA, outB]
        wid = lax.axis_index("s") * 2 + lax.axis_index("c")
        if full_sort:
            pltpu.sync_copy(s_hbm.at[0], s0)
        else:
            pltpu.sync_copy(s0_hbm.at[0], s0)

        def zero_x(j, c):
            zv = jnp.zeros((L,), jnp.float32)
            xA[pl.ds(j * L, L)] = zv
            xB[pl.ds(j * L, L)] = zv
            return c
        lax.fori_loop(0, DP // L, zero_x, 0)

        def issue_in(b, row):
            if full_sort:
                pltpu.async_copy(s_hbm.at[row], srow[b], insem[b])
            else:
                @pl.when(row == 0)
                def _():
                    pltpu.async_copy(s0_hbm.at[0], srow[b], insem[b])

                @pl.when(row > 0)
                def _():
                    pltpu.async_copy(s_hbm.at[row], srow[b], insem[b])
                    pltpu.async_copy(pold_hbm.at[row], pb[b], insem[b])

        def wait_in(b, row):
            pltpu.make_async_copy(s_hbm.at[row], srow[b], insem[b]).wait()
            if not full_sort:
                @pl.when(row > 0)
                def _():
                    pltpu.make_async_copy(
                        pold_hbm.at[row], pb[b], insem[b]).wait()

        def issue_out(b, row):
            pltpu.async_copy(xb[b], x_hbm.at[row], outsem[b])
            if full_sort:
                pltpu.async_copy(pb[b], p_hbm.at[row], outsem[b])
                pltpu.async_copy(ipb[b], ip_hbm.at[row], outsem[b])

        def wait_out(b, row):
            pltpu.make_async_copy(xb[b], x_hbm.at[row], outsem[b]).wait()
            if full_sort:
                pltpu.make_async_copy(pb[b], p_hbm.at[row], outsem[b]).wait()
                pltpu.make_async_copy(
                    ipb[b], ip_hbm.at[row], outsem[b]).wait()

        issue_in(0, wid)
        issue_in(1, NW + wid)

        def g_body(g, c):
            for b in range(2):
                row = (2 * g + b) * NW + wid
                wait_in(b, row)

                @pl.when(g > 0)
                def _(b=b, row=row):
                    wait_out(b, row)

                if full_sort:
                    @pl.when(row < N)
                    def _(b=b):
                        _sort_row(srow[b], ka, pb[b], kb, ibs, hists)
                else:
                    @pl.when(row == 0)
                    def _(b=b):
                        _sort_row(srow[b], ka, pb[b], kb, ibs, hists)

                def post_body(j, c2, b=b):
                    for u in range(4):
                        off = j * (L * 4) + u * L
                        mv = lax.iota(jnp.int32, L) + off
                        pv = pb[b][pl.ds(off, L)]
                        if full_sort:
                            plsc.store_scatter(ipb[b], [pv], mv)
                        sv = plsc.load_gather(srow[b], [pv + 1])
                        plsc.store_scatter(xb[b], [mv + 1], sv)
                        sec = plsc.load_gather(s0, [pv + 1])
                        plsc.store_scatter(xb[b], [mv + 1 + K], sec)
                    return c2
                @pl.when(row < N)
                def _(b=b, row=row):
                    lax.fori_loop(0, NV // 4, post_body, 0)
                    lane0 = lax.iota(jnp.int32, L) == 0
                    c0 = plsc.load_gather(s0,
                                          [lax.full((L,), row, jnp.int32)])
                    plsc.store_scatter(xb[b], [jnp.zeros((L,), jnp.int32)],
                                       c0, mask=lane0)

                if not full_sort:
                    @pl.when(row == 0)
                    def _(b=b):
                        def ip_body(j, c3):
                            for u in range(4):
                                off = j * (L * 4) + u * L
                                mv = lax.iota(jnp.int32, L) + off
                                pv = pb[b][pl.ds(off, L)]
                                plsc.store_scatter(ipb[b], [pv], mv)
                            return c3
                        lax.fori_loop(0, NV // 4, ip_body, 0)
                        pltpu.sync_copy(pb[b], p_hbm.at[0])
                        pltpu.sync_copy(ipb[b], ip_hbm.at[0])

                issue_out(b, row)

                @pl.when(g < RPW // 2 - 1)
                def _(b=b, row=row):
                    issue_in(b, row + 2 * NW)
            return c
        lax.fori_loop(0, RPW // 2, g_body, 0)
        for b in range(2):
            wait_out(b, (RPW - 2 + b) * NW + wid)

    return functools.partial(
        pl.kernel, out_type=out_type, mesh=_MESH, scratch_types=scratch,
        compiler_params=_SC_PARAMS)(body)


_sc_build_full = _make_sc_build(True)
_sc_build_reuse = _make_sc_build(False)


def _make_sc_bwd(qmode, split_ip):
    scratch = [
        pltpu.VMEM((DP,), jnp.float32),
        pltpu.VMEM((DP,), jnp.float32),
        pltpu.VMEM((K,), jnp.int32),
        pltpu.VMEM((K,), jnp.int32),
        pltpu.VMEM((NP,), jnp.float32),
        pltpu.VMEM((NP,), jnp.float32),
        pltpu.VMEM((NP,), jnp.float32),
        pltpu.SemaphoreType.DMA,
        pltpu.SemaphoreType.DMA,
        pltpu.SemaphoreType.DMA,
        pltpu.SemaphoreType.DMA,
    ]
    if qmode:
        out_type = (jax.ShapeDtypeStruct((1, NP), jnp.float32),
                    jax.ShapeDtypeStruct((NW, NP), jnp.float32))
    else:
        out_type = (jax.ShapeDtypeStruct((NP, NP), jnp.float32),
                    jax.ShapeDtypeStruct((NW, NP), jnp.float32))

    def body(*refs):
        if split_ip:
            (gl_hbm, ip_hbm, ip0_hbm, b_hbm, part_hbm, *sc) = refs
        else:
            (gl_hbm, ip_hbm, b_hbm, part_hbm, *sc) = refs
            ip0_hbm = None
        (glA, glB, ipbA, ipbB, brA, brB, rsum,
         inA, inB, outA, outB) = sc
        glrow = [glA, glB]
        ipbuf = [ipbA, ipbB]
        brow = [brA, brB]
        insem = [inA, inB]
        outsem = [outA, outB]
        wid = lax.axis_index("s") * 2 + lax.axis_index("c")

        def zero_body(j, c):
            zv = jnp.zeros((L,), jnp.float32)
            brA[pl.ds(j * L, L)] = zv
            brB[pl.ds(j * L, L)] = zv
            rsum[pl.ds(j * L, L)] = zv
            return c
        lax.fori_loop(0, NP // L, zero_body, 0)

        def issue_in(b, row):
            pltpu.async_copy(gl_hbm.at[row], glrow[b], insem[b])
            if split_ip:
                @pl.when(row == 0)
                def _():
                    pltpu.async_copy(ip0_hbm.at[0], ipbuf[b], insem[b])

                @pl.when(row > 0)
                def _():
                    pltpu.async_copy(ip_hbm.at[row], ipbuf[b], insem[b])
            else:
                pltpu.async_copy(ip_hbm.at[row], ipbuf[b], insem[b])

        def wait_in(b, row):
            pltpu.make_async_copy(gl_hbm.at[row], glrow[b], insem[b]).wait()
            pltpu.make_async_copy(ip_hbm.at[row], ipbuf[b], insem[b]).wait()

        issue_in(0, wid)
        issue_in(1, NW + wid)

        def g_body(g, c):
            for b in range(2):
                row = (2 * g + b) * NW + wid
                wait_in(b, row)
                if not qmode:
                    @pl.when(g > 0)
                    def _(b=b, row=row):
                        pltpu.make_async_copy(
                            brow[b], b_hbm.at[row], outsem[b]).wait()

                def gather_c(j, c2, b=b):
                    for u in range(_UR):
                        off = j * (L * _UR) + u * L
                        mv = lax.iota(jnp.int32, L) + off
                        ipv = ipbuf[b][pl.ds(off, L)]
                        cv = plsc.load_gather(glrow[b], [ipv + 1 + K])
                        old = plsc.load_gather(rsum, [mv + 1])
                        plsc.store_scatter(rsum, [mv + 1], old + cv)
                    return c2

                @pl.when(row < N)
                def _(b=b, row=row):
                    lax.fori_loop(0, NV // _UR, gather_c, 0)

                def gather_a(j, c2, b=b):
                    for u in range(_UR):
                        off = j * (L * _UR) + u * L
                        mv = lax.iota(jnp.int32, L) + off
                        ipv = ipbuf[b][pl.ds(off, L)]
                        av = plsc.load_gather(glrow[b], [ipv + 1])
                        plsc.store_scatter(brow[b], [mv + 1], av)
                    return c2

                if qmode:
                    @pl.when(row == 0)
                    def _(b=b):
                        lax.fori_loop(0, NV // _UR, gather_a, 0)
                        pltpu.sync_copy(brow[b], b_hbm.at[0])
                else:
                    @pl.when(row < N)
                    def _(b=b):
                        lax.fori_loop(0, NV // _UR, gather_a, 0)
                    pltpu.async_copy(brow[b], b_hbm.at[row], outsem[b])

                @pl.when(g < RPW // 2 - 1)
                def _(b=b, row=row):
                    issue_in(b, row + 2 * NW)
            return c
        lax.fori_loop(0, RPW // 2, g_body, 0)
        if not qmode:
            for b in range(2):
                pltpu.make_async_copy(
                    brow[b], b_hbm.at[(RPW - 2 + b) * NW + wid],
                    outsem[b]).wait()
        pltpu.sync_copy(rsum, part_hbm.at[wid])

    return functools.partial(
        pl.kernel, out_type=out_type, mesh=_MESH, scratch_types=scratch,
        compiler_params=_SC_PARAMS)(body)


_sc_bwd_q = _make_sc_bwd(True, False)
_sc_bwd_d = _make_sc_bwd(False, True)


def _bwd_q_row_body(a_ref, part_ref, glb_ref, f_ref, n_ref, feat_ref,
                    fq_ref, s0_ref, f0_ref, n0_ref):
    rowsum = jnp.sum(part_ref[...], axis=0, keepdims=True)
    t = a_ref[...] + rowsum
    F = f_ref[...]
    glcol0 = glb_ref[...][:, 0:1]
    dF0 = lax.dot_general(t, F, (((1,), (0,)), ((), ())),
                          preferred_element_type=jnp.float32)
    dF0 += lax.dot_general(glcol0, F, (((0,), (0,)), ((), ())),
                           preferred_element_type=jnp.float32)
    dF0 += glb_ref[...][0:1, 0:1] * F[0:1]
    F0old = F[0:1]
    n0old = n_ref[...][0:1]
    dot = jnp.sum(dF0 * F0old, axis=1, keepdims=True)
    g0 = (dF0 - jnp.where(n0old > EPS_N, dot * F0old, 0.0)) \
        / jnp.maximum(n0old, EPS_N)
    newfq = feat_ref[0:1, :] - LR * g0
    n0 = jnp.sqrt(jnp.sum(newfq * newfq, axis=1, keepdims=True))
    F0 = newfq / jnp.maximum(n0, EPS_N)
    s0 = lax.dot_general(F0, F, (((1,), (1,)), ((), ())),
                         preferred_element_type=jnp.float32)
    selfsim = jnp.sum(F0 * F0, axis=1, keepdims=True)
    ci = lax.broadcasted_iota(jnp.int32, (1, NP), 1)
    fq_ref[...] = newfq
    s0_ref[...] = jnp.where(ci == 0, selfsim, s0)
    f0_ref[...] = F0
    n0_ref[...] = n0


def _bwd_q_row(arow0, partials, glb, F, n, feat_p):
    return pl.pallas_call(
        _bwd_q_row_body,
        out_shape=(
            jax.ShapeDtypeStruct((1, FEAT), jnp.float32),
            jax.ShapeDtypeStruct((1, NP), jnp.float32),
            jax.ShapeDtypeStruct((1, FEAT), jnp.float32),
            jax.ShapeDtypeStruct((1, 1), jnp.float32),
        ),
    )(arow0, partials, glb, F, n, feat_p)


def _bwd_d_common(b_ref, part_ref, glb_ref, f_ref, n_ref, feat_ref,
                  fq_ref, f0_ref, n0_ref):
    ri1 = lax.broadcasted_iota(jnp.int32, (NP, 1), 0)
    rif = lax.broadcasted_iota(jnp.int32, (NP, FEAT), 0)
    F = jnp.where(rif == 0, f0_ref[...], f_ref[...])
    n = jnp.where(ri1 == 0, n0_ref[...], n_ref[...])
    feat = jnp.where(rif == 0, fq_ref[...], feat_ref[...])
    rowsum = jnp.sum(part_ref[...], axis=0, keepdims=True)
    glcol0 = glb_ref[...][:, 0:1]
    B = b_ref[...]
    ri = lax.broadcasted_iota(jnp.int32, (NP, NP), 0)
    ci = lax.broadcasted_iota(jnp.int32, (NP, NP), 1)
    M = B + jnp.where(ci == 0, glcol0, 0.0) + jnp.where(ri == 0, rowsum, 0.0)
    Mb = M.astype(jnp.bfloat16)
    Fb = F.astype(jnp.bfloat16)
    dF = lax.dot_general(Mb, Fb, (((1,), (0,)), ((), ())),
                         preferred_element_type=jnp.float32)
    dF += lax.dot_general(Mb, Fb, (((0,), (0,)), ((), ())),
                          preferred_element_type=jnp.float32)
    dot = jnp.sum(dF * F, axis=1, keepdims=True)
    g = (dF - jnp.where(n > EPS_N, dot * F, 0.0)) / jnp.maximum(n, EPS_N)
    upd = (rif >= 1) & (rif < N)
    return feat - LR * jnp.where(upd, g, 0.0)


def _bwd_d_sim_body(b_ref, part_ref, glb_ref, f_ref, n_ref, feat_ref,
                    fq_ref, f0_ref, n0_ref, featout_ref, s_ref, fout_ref,
                    nout_ref):
    feat_new = _bwd_d_common(b_ref, part_ref, glb_ref, f_ref, n_ref,
                             feat_ref, fq_ref, f0_ref, n0_ref)
    featout_ref[...] = feat_new
    n2 = jnp.sqrt(jnp.sum(feat_new * feat_new, axis=1, keepdims=True))
    F2 = feat_new / jnp.maximum(n2, EPS_N)
    s_ref[...] = lax.dot_general(F2, F2, (((1,), (1,)), ((), ())),
                                 preferred_element_type=jnp.float32)
    fout_ref[...] = F2
    nout_ref[...] = n2


def _bwd_d_sim(Bmat, partials, glb, F, n, feat_p, fq, f0, n0):
    return pl.pallas_call(
        _bwd_d_sim_body,
        out_shape=(
            jax.ShapeDtypeStruct((NP, FEAT), jnp.float32),
            jax.ShapeDtypeStruct((NP, NP), jnp.float32),
            jax.ShapeDtypeStruct((NP, FEAT), jnp.float32),
            jax.ShapeDtypeStruct((NP, 1), jnp.float32),
        ),
    )(Bmat, partials, glb, F, n, feat_p, fq, f0, n0)


def _bwd_d_final_body(b_ref, part_ref, glb_ref, f_ref, n_ref, feat_ref,
                      fq_ref, f0_ref, n0_ref, out_ref):
    feat_new = _bwd_d_common(b_ref, part_ref, glb_ref, f_ref, n_ref,
                             feat_ref, fq_ref, f0_ref, n0_ref)
    n2 = jnp.sqrt(jnp.sum(feat_new * feat_new, axis=1, keepdims=True))
    F2 = feat_new / jnp.maximum(n2, EPS_N)
    out_ref[...] = lax.dot_general(F2[0:1], F2, (((1,), (1,)), ((), ())),
                                   preferred_element_type=jnp.float32)


def _bwd_d_final(Bmat, partials, glb, F, n, feat_p, fq, f0, n0):
    return pl.pallas_call(
        _bwd_d_final_body,
        out_shape=jax.ShapeDtypeStruct((1, NP), jnp.float32),
    )(Bmat, partials, glb, F, n, feat_p, fq, f0, n0)


def _prep_params(p):
    scal = jnp.stack([p['in1_w'], p['in1_b'], p['in2_w'], p['in2_b'],
                      p['in3_w'], p['in3_b']])
    w1 = jnp.pad(p['l1_W'], ((0, 0), (0, DP - DIN))).astype(jnp.bfloat16)
    w2 = p['l2_W'].astype(jnp.bfloat16)
    w3 = jnp.pad(p['l3_W'], ((0, DP - DIN), (0, 0))).astype(jnp.bfloat16)
    b1 = p['l1_b'][None, :]
    b2 = p['l2_b'][None, :]
    b3 = jnp.pad(p['l3_b'], (0, DP - DIN))[None, :]
    return dict(scal=scal, w1=w1, b1=b1, w2=w2, b2=b2, w3=w3, b3=b3)


def _half_round(S, F, n, feat_p, wq, wd):
    x1, p1, ip1 = _sc_build_full(S)
    gl1 = _dni(x1, wq)
    glb1 = lax.slice(gl1, (0, 0), (NP, 128))
    a1, pt1 = _sc_bwd_q(gl1, ip1)
    fq, s0n, f0, n0 = _bwd_q_row(a1, pt1, glb1, F, n, feat_p)
    x2, _, ip02 = _sc_build_reuse(S, s0n, p1)
    gl2 = _dni(x2, wd)
    glb2 = lax.slice(gl2, (0, 0), (NP, 128))
    B2, pt2 = _sc_bwd_d(gl2, ip1, ip02)
    return (B2, pt2, glb2, F, n, feat_p, fq, f0, n0)


def kernel(feat_query, feat_database, params):
    feat = jnp.concatenate([feat_query, feat_database], axis=0)
    feat_p = jnp.pad(feat, ((0, NP - N), (0, 0)))
    wq = _prep_params(params['query'])
    wd = _prep_params(params['database'])
    S1, F1, n1 = _similarity(feat_p)
    args = _half_round(S1, F1, n1, feat_p, wq, wd)
    feat2, S2, F2, n2 = _bwd_d_sim(*args)
    args = _half_round(S2, F2, n2, feat2, wq, wd)
    out = _bwd_d_final(*args)
    return out[0, 1:N]


# speedup vs baseline: 178.7397x; 1.2012x over previous
import functools

import jax
import jax.numpy as jnp
from jax import lax
from jax.experimental import pallas as pl
from jax.experimental.pallas import tpu as pltpu
from jax.experimental.pallas import tpu_sc as plsc

FEAT = 512
K = 1024
N = K + 1
NP = 1152
DIN = 2 * K + 1
DP = 2176
HID = 2048
LR = 1e-3
EPS_N = 1e-12
EPS_IN = 1e-5

L = 16
NV = K // L
NB = 256
NW = 32
RPW = NP // NW

_MESH = plsc.VectorSubcoreMesh(core_axis_name="c", subcore_axis_name="s")
_SC_PARAMS = pltpu.CompilerParams(needs_layout_passes=False)


def _sim_body(feat_ref, s_ref, f_ref, n_ref):
    x = feat_ref[...]
    n = jnp.sqrt(jnp.sum(x * x, axis=1, keepdims=True))
    F = x / jnp.maximum(n, EPS_N)
    s_ref[...] = lax.dot_general(
        F, F, (((1,), (1,)), ((), ())), preferred_element_type=jnp.float32)
    f_ref[...] = F
    n_ref[...] = n


def _similarity(feat_p):
    return pl.pallas_call(
        _sim_body,
        out_shape=(
            jax.ShapeDtypeStruct((NP, NP), jnp.float32),
            jax.ShapeDtypeStruct((NP, FEAT), jnp.float32),
            jax.ShapeDtypeStruct((NP, 1), jnp.float32),
        ),
    )(feat_p)


def _final_body(feat_ref, out_ref):
    x = feat_ref[...]
    n = jnp.sqrt(jnp.sum(x * x, axis=1, keepdims=True))
    F = x / jnp.maximum(n, EPS_N)
    out_ref[...] = lax.dot_general(
        F[0:1], F, (((1,), (1,)), ((), ())),
        preferred_element_type=jnp.float32)


def _final_row(feat_p):
    return pl.pallas_call(
        _final_body,
        out_shape=jax.ShapeDtypeStruct((1, NP), jnp.float32),
    )(feat_p)


def _dni_body(scal_ref, x_ref, w1_ref, b1_ref, w2_ref, b2_ref, w3_ref,
              b3_ref, out_ref):
    x = x_ref[...]
    mask = lax.broadcasted_iota(jnp.int32, x.shape, 1) < DIN

    def inorm(h, d, w, b, msk=None):
        mean = jnp.sum(h, axis=1, keepdims=True) / d
        var = jnp.sum(h * h, axis=1, keepdims=True) / d - mean * mean
        out = (h - mean) * lax.rsqrt(var + EPS_IN) * w + b
        if msk is not None:
            out = jnp.where(msk, out, 0.0)
        return out

    def mm(a, w_ref):
        return lax.dot_general(
            a.astype(jnp.bfloat16), w_ref[...],
            (((1,), (1,)), ((), ())), preferred_element_type=jnp.float32)

    h = inorm(x, DIN, scal_ref[0], scal_ref[1], mask)
    h = jnp.maximum(mm(h, w1_ref) + b1_ref[...], 0.0)
    h = inorm(h, HID, scal_ref[2], scal_ref[3])
    h = jnp.maximum(mm(h, w2_ref) + b2_ref[...], 0.0)
    h = inorm(h, HID, scal_ref[4], scal_ref[5])
    out_ref[...] = mm(h, w3_ref) + b3_ref[...]


def _dni(x_p, wp):
    br = 256
    w_spec = lambda shape: pl.BlockSpec(shape, lambda i: (0, 0))
    return pl.pallas_call(
        _dni_body,
        grid=(NP // br,),
        in_specs=[
            pl.BlockSpec(memory_space=pltpu.SMEM),
            pl.BlockSpec((br, DP), lambda i: (i, 0)),
            w_spec((HID, DP)), w_spec((1, HID)),
            w_spec((HID, HID)), w_spec((1, HID)),
            w_spec((DP, HID)), w_spec((1, DP)),
        ],
        out_specs=pl.BlockSpec((br, DP), lambda i: (i, 0)),
        out_shape=jax.ShapeDtypeStruct((NP, DP), jnp.float32),
    )(wp['scal'], x_p, wp['w1'], wp['b1'], wp['w2'], wp['b2'],
      wp['w3'], wp['b3'])


def _monotone_desc(u):
    m = lax.shift_right_arithmetic(u, 31) | jnp.int32(-2 ** 31)
    return ~(u ^ m)


def _digit(k, shift):
    return lax.shift_right_logical(k, shift) & 0xFF


_SHIFTS = (0, 8, 16, 24)
_UR = 8


def _sort_row(srow, ka, ia, kb, ib, hists):
    zv = jnp.zeros((L,), jnp.int32)
    for h in hists:
        for b in range(NB // L):
            h[pl.ds(b * L, L)] = zv

    ones = jnp.full((L,), 1, jnp.int32)

    def lh_body(j, c):
        for u in range(_UR):
            off = j * (L * _UR) + u * L
            kv = plsc.load_gather(srow, [lax.iota(jnp.int32, L) + (off + 1)])
            m = _monotone_desc(plsc.bitcast(kv, jnp.int32))
            ka[pl.ds(off, L)] = m
            for pn, shift in enumerate(_SHIFTS):
                plsc.addupdate_scatter(hists[pn], [_digit(m, shift)], ones)
        return c
    lax.fori_loop(0, NV // _UR, lh_body, 0)

    bufs = [(ka, ia), (kb, ib)]
    for pn, shift in enumerate(_SHIFTS):
        hist = hists[pn]
        carry = 0
        for b in range(NB // L):
            v = hist[pl.ds(b * L, L)]
            cs = plsc.cumsum(v)
            hist[pl.ds(b * L, L)] = cs - v + carry
            carry = carry + jnp.sum(v)
        src_k, src_i = bufs[pn % 2]
        dst_k, dst_i = bufs[(pn + 1) % 2]

        def perm_body(j, c, pn=pn, shift=shift, hist=hist, src_k=src_k,
                      src_i=src_i, dst_k=dst_k, dst_i=dst_i):
            for u in range(_UR):
                off = j * (L * _UR) + u * L
                k = src_k[pl.ds(off, L)]
                if pn == 0:
                    iv = lax.iota(jnp.int32, L) + off
                else:
                    iv = src_i[pl.ds(off, L)]
                d = _digit(k, shift)
                base = plsc.load_gather(hist, [d])
                occ, lastm = plsc.scan_count(d)
                pos = base + occ - 1
                plsc.store_scatter(dst_k, [pos], k)
                plsc.store_scatter(dst_i, [pos], iv)
                plsc.addupdate_scatter(hist, [d], occ, mask=lastm)
            return c
        lax.fori_loop(0, NV // _UR, perm_body, 0)


def _make_sc_build(full_sort, nbuf=2):
    scratch = (
        [pltpu.VMEM((NP,), jnp.float32)] * nbuf
        + [pltpu.VMEM((NP,), jnp.float32)]
        + [pltpu.VMEM((DP,), jnp.float32)] * nbuf
        + [pltpu.VMEM((K,), jnp.int32)] * nbuf
        + [pltpu.VMEM((K,), jnp.int32)] * nbuf
        + [pltpu.VMEM((K,), jnp.int32)] * 3
        + [pltpu.VMEM((NB,), jnp.int32)] * 4
        + [pltpu.SemaphoreType.DMA] * (2 * nbuf)
    )
    if full_sort:
        out_type = (
            jax.ShapeDtypeStruct((NP, DP), jnp.float32),
            jax.ShapeDtypeStruct((NP, K), jnp.int32),
            jax.ShapeDtypeStruct((NP, K), jnp.int32),
        )
    else:
        out_type = (
            jax.ShapeDtypeStruct((NP, DP), jnp.float32),
            jax.ShapeDtypeStruct((1, K), jnp.int32),
            jax.ShapeDtypeStruct((1, K), jnp.int32),
        )

    def body(*refs):
        if full_sort:
            (s_hbm, x_hbm, p_hbm, ip_hbm, *sc) = refs
            pold_hbm = None
        else:
            (s_hbm, s0_hbm, pold_hbm, x_hbm, p_hbm, ip_hbm, *sc) = refs
        sc = list(sc)
        srow = sc[:nbuf]
        s0 = sc[nbuf]
        xb = sc[nbuf + 1:2 * nbuf + 1]
        pb = sc[2 * nbuf + 1:3 * nbuf + 1]
        ipb = sc[3 * nbuf + 1:4 * nbuf + 1]
        ka, kb, ibs = sc[4 * nbuf + 1:4 * nbuf + 4]
        hists = sc[4 * nbuf + 4:4 * nbuf + 8]
        insem = sc[4 * nbuf + 8:5 * nbuf + 8]
        outsem = sc[5 * nbuf + 8:6 * nbuf + 8]
        wid = lax.axis_index("s") * 2 + lax.axis_index("c")
        if full_sort:
            pltpu.sync_copy(s_hbm.at[0], s0)
        else:
            pltpu.sync_copy(s0_hbm.at[0], s0)

        def zero_x(j, c):
            zv = jnp.zeros((L,), jnp.float32)
            for xbb in xb:
                xbb[pl.ds(j * L, L)] = zv
            return c
        lax.fori_loop(0, DP // L, zero_x, 0)

        def issue_in(b, row):
            if full_sort:
                pltpu.async_copy(s_hbm.at[row], srow[b], insem[b])
            else:
                @pl.when(row == 0)
                def _():
                    pltpu.async_copy(s0_hbm.at[0], srow[b], insem[b])

                @pl.when(row > 0)
                def _():
                    pltpu.async_copy(s_hbm.at[row], srow[b], insem[b])
                    pltpu.async_copy(pold_hbm.at[row], pb[b], insem[b])

        def wait_in(b, row):
            pltpu.make_async_copy(s_hbm.at[row], srow[b], insem[b]).wait()
            if not full_sort:
                @pl.when(row > 0)
                def _():
                    pltpu.make_async_copy(
                        pold_hbm.at[row], pb[b], insem[b]).wait()

        def issue_out(b, row):
            pltpu.async_copy(xb[b], x_hbm.at[row], outsem[b])
            if full_sort:
                pltpu.async_copy(pb[b], p_hbm.at[row], outsem[b])
                pltpu.async_copy(ipb[b], ip_hbm.at[row], outsem[b])

        def wait_out(b, row):
            pltpu.make_async_copy(xb[b], x_hbm.at[row], outsem[b]).wait()
            if full_sort:
                pltpu.make_async_copy(pb[b], p_hbm.at[row], outsem[b]).wait()
                pltpu.make_async_copy(
                    ipb[b], ip_hbm.at[row], outsem[b]).wait()

        for b in range(nbuf):
            issue_in(b, b * NW + wid)

        def g_body(g, c):
            for b in range(nbuf):
                row = (nbuf * g + b) * NW + wid
                wait_in(b, row)

                @pl.when(g > 0)
                def _(b=b, row=row):
                    wait_out(b, row)

                if full_sort:
                    @pl.when(row < N)
                    def _(b=b):
                        _sort_row(srow[b], ka, pb[b], kb, ibs, hists)
                else:
                    @pl.when(row == 0)
                    def _(b=b):
                        _sort_row(srow[b], ka, pb[b], kb, ibs, hists)

                def post_body(j, c2, b=b):
                    for u in range(4):
                        off = j * (L * 4) + u * L
                        mv = lax.iota(jnp.int32, L) + off
                        pv = pb[b][pl.ds(off, L)]
                        if full_sort:
                            plsc.store_scatter(ipb[b], [pv], mv)
                        sv = plsc.load_gather(srow[b], [pv + 1])
                        plsc.store_scatter(xb[b], [mv + 1], sv)
                        sec = plsc.load_gather(s0, [pv + 1])
                        plsc.store_scatter(xb[b], [mv + 1 + K], sec)
                    return c2
                @pl.when(row < N)
                def _(b=b, row=row):
                    lax.fori_loop(0, NV // 4, post_body, 0)
                    lane0 = lax.iota(jnp.int32, L) == 0
                    c0 = plsc.load_gather(s0,
                                          [lax.full((L,), row, jnp.int32)])
                    plsc.store_scatter(xb[b], [jnp.zeros((L,), jnp.int32)],
                                       c0, mask=lane0)

                if not full_sort:
                    @pl.when(row == 0)
                    def _(b=b):
                        def ip_body(j, c3):
                            for u in range(4):
                                off = j * (L * 4) + u * L
                                mv = lax.iota(jnp.int32, L) + off
                                pv = pb[b][pl.ds(off, L)]
                                plsc.store_scatter(ipb[b], [pv], mv)
                            return c3
                        lax.fori_loop(0, NV // 4, ip_body, 0)
                        pltpu.sync_copy(pb[b], p_hbm.at[0])
                        pltpu.sync_copy(ipb[b], ip_hbm.at[0])

                issue_out(b, row)

                @pl.when(g < RPW // nbuf - 1)
                def _(b=b, row=row):
                    issue_in(b, row + nbuf * NW)
            return c
        lax.fori_loop(0, RPW // nbuf, g_body, 0)
        for b in range(nbuf):
            wait_out(b, (RPW - nbuf + b) * NW + wid)

    return functools.partial(
        pl.kernel, out_type=out_type, mesh=_MESH, scratch_types=scratch,
        compiler_params=_SC_PARAMS)(body)


_sc_build_full = _make_sc_build(True, nbuf=2)
_sc_build_reuse = _make_sc_build(False, nbuf=3)


def _make_sc_bwd(qmode, split_ip, nbuf=3):
    scratch = (
        [pltpu.VMEM((DP,), jnp.float32)] * nbuf
        + [pltpu.VMEM((K,), jnp.int32)] * nbuf
        + [pltpu.VMEM((NP,), jnp.float32)] * nbuf
        + [pltpu.VMEM((NP,), jnp.float32)]
        + [pltpu.SemaphoreType.DMA] * (2 * nbuf)
    )
    if qmode:
        out_type = (jax.ShapeDtypeStruct((1, NP), jnp.float32),
                    jax.ShapeDtypeStruct((NW, NP), jnp.float32))
    else:
        out_type = (jax.ShapeDtypeStruct((NP, NP), jnp.float32),
                    jax.ShapeDtypeStruct((NW, NP), jnp.float32))

    def body(*refs):
        if split_ip:
            (gl_hbm, ip_hbm, ip0_hbm, b_hbm, part_hbm, *sc) = refs
        else:
            (gl_hbm, ip_hbm, b_hbm, part_hbm, *sc) = refs
            ip0_hbm = None
        sc = list(sc)
        glrow = sc[:nbuf]
        ipbuf = sc[nbuf:2 * nbuf]
        brow = sc[2 * nbuf:3 * nbuf]
        rsum = sc[3 * nbuf]
        insem = sc[3 * nbuf + 1:4 * nbuf + 1]
        outsem = sc[4 * nbuf + 1:5 * nbuf + 1]
        wid = lax.axis_index("s") * 2 + lax.axis_index("c")

        def zero_body(j, c):
            zv = jnp.zeros((L,), jnp.float32)
            for brb in brow:
                brb[pl.ds(j * L, L)] = zv
            rsum[pl.ds(j * L, L)] = zv
            return c
        lax.fori_loop(0, NP // L, zero_body, 0)

        def issue_in(b, row):
            pltpu.async_copy(gl_hbm.at[row], glrow[b], insem[b])
            if split_ip:
                @pl.when(row == 0)
                def _():
                    pltpu.async_copy(ip0_hbm.at[0], ipbuf[b], insem[b])

                @pl.when(row > 0)
                def _():
                    pltpu.async_copy(ip_hbm.at[row], ipbuf[b], insem[b])
            else:
                pltpu.async_copy(ip_hbm.at[row], ipbuf[b], insem[b])

        def wait_in(b, row):
            pltpu.make_async_copy(gl_hbm.at[row], glrow[b], insem[b]).wait()
            pltpu.make_async_copy(ip_hbm.at[row], ipbuf[b], insem[b]).wait()

        for b in range(nbuf):
            issue_in(b, b * NW + wid)

        def g_body(g, c):
            for b in range(nbuf):
                row = (nbuf * g + b) * NW + wid
                wait_in(b, row)
                if not qmode:
                    @pl.when(g > 0)
                    def _(b=b, row=row):
                        pltpu.make_async_copy(
                            brow[b], b_hbm.at[row], outsem[b]).wait()

                def gather_c(j, c2, b=b):
                    for u in range(_UR):
                        off = j * (L * _UR) + u * L
                        mv = lax.iota(jnp.int32, L) + off
                        ipv = ipbuf[b][pl.ds(off, L)]
                        cv = plsc.load_gather(glrow[b], [ipv + 1 + K])
                        old = plsc.load_gather(rsum, [mv + 1])
                        plsc.store_scatter(rsum, [mv + 1], old + cv)
                    return c2

                @pl.when(row < N)
                def _(b=b, row=row):
                    lax.fori_loop(0, NV // _UR, gather_c, 0)

                def gather_a(j, c2, b=b):
                    for u in range(_UR):
                        off = j * (L * _UR) + u * L
                        mv = lax.iota(jnp.int32, L) + off
                        ipv = ipbuf[b][pl.ds(off, L)]
                        av = plsc.load_gather(glrow[b], [ipv + 1])
                        plsc.store_scatter(brow[b], [mv + 1], av)
                    return c2

                if qmode:
                    @pl.when(row == 0)
                    def _(b=b):
                        lax.fori_loop(0, NV // _UR, gather_a, 0)
                        pltpu.sync_copy(brow[b], b_hbm.at[0])
                else:
                    @pl.when(row < N)
                    def _(b=b):
                        lax.fori_loop(0, NV // _UR, gather_a, 0)
                    pltpu.async_copy(brow[b], b_hbm.at[row], outsem[b])

                @pl.when(g < RPW // nbuf - 1)
                def _(b=b, row=row):
                    issue_in(b, row + nbuf * NW)
            return c
        lax.fori_loop(0, RPW // nbuf, g_body, 0)
        if not qmode:
            for b in range(nbuf):
                pltpu.make_async_copy(
                    brow[b], b_hbm.at[(RPW - nbuf + b) * NW + wid],
                    outsem[b]).wait()
        pltpu.sync_copy(rsum, part_hbm.at[wid])

    return functools.partial(
        pl.kernel, out_type=out_type, mesh=_MESH, scratch_types=scratch,
        compiler_params=_SC_PARAMS)(body)


_sc_bwd_q = _make_sc_bwd(True, False, nbuf=3)
_sc_bwd_d = _make_sc_bwd(False, True, nbuf=3)


def _bwd_q_row_body(a_ref, part_ref, glb_ref, f_ref, n_ref, feat_ref,
                    fq_ref, s0_ref, f0_ref, n0_ref):
    rowsum = jnp.sum(part_ref[...], axis=0, keepdims=True)
    t = a_ref[...] + rowsum
    F = f_ref[...]
    glcol0 = glb_ref[...][:, 0:1]
    dF0 = lax.dot_general(t, F, (((1,), (0,)), ((), ())),
                          preferred_element_type=jnp.float32)
    dF0 += lax.dot_general(glcol0, F, (((0,), (0,)), ((), ())),
                           preferred_element_type=jnp.float32)
    dF0 += glb_ref[...][0:1, 0:1] * F[0:1]
    F0old = F[0:1]
    n0old = n_ref[...][0:1]
    dot = jnp.sum(dF0 * F0old, axis=1, keepdims=True)
    g0 = (dF0 - jnp.where(n0old > EPS_N, dot * F0old, 0.0)) \
        / jnp.maximum(n0old, EPS_N)
    newfq = feat_ref[0:1, :] - LR * g0
    n0 = jnp.sqrt(jnp.sum(newfq * newfq, axis=1, keepdims=True))
    F0 = newfq / jnp.maximum(n0, EPS_N)
    s0 = lax.dot_general(F0, F, (((1,), (1,)), ((), ())),
                         preferred_element_type=jnp.float32)
    selfsim = jnp.sum(F0 * F0, axis=1, keepdims=True)
    ci = lax.broadcasted_iota(jnp.int32, (1, NP), 1)
    fq_ref[...] = newfq
    s0_ref[...] = jnp.where(ci == 0, selfsim, s0)
    f0_ref[...] = F0
    n0_ref[...] = n0


def _bwd_q_row(arow0, partials, glb, F, n, feat_p):
    return pl.pallas_call(
        _bwd_q_row_body,
        out_shape=(
            jax.ShapeDtypeStruct((1, FEAT), jnp.float32),
            jax.ShapeDtypeStruct((1, NP), jnp.float32),
            jax.ShapeDtypeStruct((1, FEAT), jnp.float32),
            jax.ShapeDtypeStruct((1, 1), jnp.float32),
        ),
    )(arow0, partials, glb, F, n, feat_p)


def _bwd_d_common(b_ref, part_ref, glb_ref, f_ref, n_ref, feat_ref,
                  fq_ref, f0_ref, n0_ref):
    ri1 = lax.broadcasted_iota(jnp.int32, (NP, 1), 0)
    rif = lax.broadcasted_iota(jnp.int32, (NP, FEAT), 0)
    F = jnp.where(rif == 0, f0_ref[...], f_ref[...])
    n = jnp.where(ri1 == 0, n0_ref[...], n_ref[...])
    feat = jnp.where(rif == 0, fq_ref[...], feat_ref[...])
    rowsum = jnp.sum(part_ref[...], axis=0, keepdims=True)
    glcol0 = glb_ref[...][:, 0:1]
    B = b_ref[...]
    ri = lax.broadcasted_iota(jnp.int32, (NP, NP), 0)
    ci = lax.broadcasted_iota(jnp.int32, (NP, NP), 1)
    M = B + jnp.where(ci == 0, glcol0, 0.0) + jnp.where(ri == 0, rowsum, 0.0)
    Mb = M.astype(jnp.bfloat16)
    Fb = F.astype(jnp.bfloat16)
    dF = lax.dot_general(Mb, Fb, (((1,), (0,)), ((), ())),
                         preferred_element_type=jnp.float32)
    dF += lax.dot_general(Mb, Fb, (((0,), (0,)), ((), ())),
                          preferred_element_type=jnp.float32)
    dot = jnp.sum(dF * F, axis=1, keepdims=True)
    g = (dF - jnp.where(n > EPS_N, dot * F, 0.0)) / jnp.maximum(n, EPS_N)
    upd = (rif >= 1) & (rif < N)
    return feat - LR * jnp.where(upd, g, 0.0)


def _bwd_d_sim_body(b_ref, part_ref, glb_ref, f_ref, n_ref, feat_ref,
                    fq_ref, f0_ref, n0_ref, featout_ref, s_ref, fout_ref,
                    nout_ref):
    feat_new = _bwd_d_common(b_ref, part_ref, glb_ref, f_ref, n_ref,
                             feat_ref, fq_ref, f0_ref, n0_ref)
    featout_ref[...] = feat_new
    n2 = jnp.sqrt(jnp.sum(feat_new * feat_new, axis=1, keepdims=True))
    F2 = feat_new / jnp.maximum(n2, EPS_N)
    s_ref[...] = lax.dot_general(F2, F2, (((1,), (1,)), ((), ())),
                                 preferred_element_type=jnp.float32)
    fout_ref[...] = F2
    nout_ref[...] = n2


def _bwd_d_sim(Bmat, partials, glb, F, n, feat_p, fq, f0, n0):
    return pl.pallas_call(
        _bwd_d_sim_body,
        out_shape=(
            jax.ShapeDtypeStruct((NP, FEAT), jnp.float32),
            jax.ShapeDtypeStruct((NP, NP), jnp.float32),
            jax.ShapeDtypeStruct((NP, FEAT), jnp.float32),
            jax.ShapeDtypeStruct((NP, 1), jnp.float32),
        ),
    )(Bmat, partials, glb, F, n, feat_p, fq, f0, n0)


def _bwd_d_final_body(b_ref, part_ref, glb_ref, f_ref, n_ref, feat_ref,
                      fq_ref, f0_ref, n0_ref, out_ref):
    feat_new = _bwd_d_common(b_ref, part_ref, glb_ref, f_ref, n_ref,
                             feat_ref, fq_ref, f0_ref, n0_ref)
    n2 = jnp.sqrt(jnp.sum(feat_new * feat_new, axis=1, keepdims=True))
    F2 = feat_new / jnp.maximum(n2, EPS_N)
    out_ref[...] = lax.dot_general(F2[0:1], F2, (((1,), (1,)), ((), ())),
                                   preferred_element_type=jnp.float32)


def _bwd_d_final(Bmat, partials, glb, F, n, feat_p, fq, f0, n0):
    return pl.pallas_call(
        _bwd_d_final_body,
        out_shape=jax.ShapeDtypeStruct((1, NP), jnp.float32),
    )(Bmat, partials, glb, F, n, feat_p, fq, f0, n0)


def _prep_params(p):
    scal = jnp.stack([p['in1_w'], p['in1_b'], p['in2_w'], p['in2_b'],
                      p['in3_w'], p['in3_b']])
    w1 = jnp.pad(p['l1_W'], ((0, 0), (0, DP - DIN))).astype(jnp.bfloat16)
    w2 = p['l2_W'].astype(jnp.bfloat16)
    w3 = jnp.pad(p['l3_W'], ((0, DP - DIN), (0, 0))).astype(jnp.bfloat16)
    b1 = p['l1_b'][None, :]
    b2 = p['l2_b'][None, :]
    b3 = jnp.pad(p['l3_b'], (0, DP - DIN))[None, :]
    return dict(scal=scal, w1=w1, b1=b1, w2=w2, b2=b2, w3=w3, b3=b3)


def _half_round(S, F, n, feat_p, wq, wd):
    x1, p1, ip1 = _sc_build_full(S)
    gl1 = _dni(x1, wq)
    glb1 = lax.slice(gl1, (0, 0), (NP, 128))
    a1, pt1 = _sc_bwd_q(gl1, ip1)
    fq, s0n, f0, n0 = _bwd_q_row(a1, pt1, glb1, F, n, feat_p)
    x2, _, ip02 = _sc_build_reuse(S, s0n, p1)
    gl2 = _dni(x2, wd)
    glb2 = lax.slice(gl2, (0, 0), (NP, 128))
    B2, pt2 = _sc_bwd_d(gl2, ip1, ip02)
    return (B2, pt2, glb2, F, n, feat_p, fq, f0, n0)


def kernel(feat_query, feat_database, params):
    feat = jnp.concatenate([feat_query, feat_database], axis=0)
    feat_p = jnp.pad(feat, ((0, NP - N), (0, 0)))
    wq = _prep_params(params['query'])
    wd = _prep_params(params['database'])
    S1, F1, n1 = _similarity(feat_p)
    args = _half_round(S1, F1, n1, feat_p, wq, wd)
    feat2, S2, F2, n2 = _bwd_d_sim(*args)
    args = _half_round(S2, F2, n2, feat2, wq, wd)
    out = _bwd_d_final(*args)
    return out[0, 1:N]
